# Initial kernel scaffold; baseline (speedup 1.0000x reference)
import jax
import jax.numpy as jnp
from jax.experimental import pallas as pl


def kernel(x, edge_index, e, xbatch, bn_node_g, bn_node_b, bn_edge_g, bn_edge_b, nn1_W0, nn1_b0, nn1_W1, nn1_b1, conv1_root, conv1_bias, nn2_W0, nn2_b0, nn2_W1, nn2_b1, conv2_root, conv2_bias, mlp_W0, mlp_b0, mlp_W1, mlp_b1, mlp_W2, mlp_b2, mlp_W3, mlp_b3, mlp_W4, mlp_b4):
    raise NotImplementedError("write your pallas kernel here")


# speedup vs baseline: 1.0907x; 1.0907x over previous
import functools

import jax
import jax.numpy as jnp
from jax import lax
from jax.experimental import pallas as pl
from jax.experimental.pallas import tpu as pltpu
from jax.experimental.pallas import tpu_sc as plsc

_NC, _NS = 2, 16
_NW = _NC * _NS
_CH = 128


_XLA_SCATTER = False
_XLA_GATHER = False


def _leaky(v):
    return jnp.where(v >= 0, v, 0.1 * v)


def _estats_body(e_ref, s_ref, q_ref):
    i = pl.program_id(0)

    @pl.when(i == 0)
    def _init():
        s_ref[...] = jnp.zeros_like(s_ref)
        q_ref[...] = jnp.zeros_like(q_ref)

    blk = e_ref[...]
    s_ref[...] += jnp.sum(blk, axis=0, keepdims=True)
    q_ref[...] += jnp.sum(blk * blk, axis=0, keepdims=True)


def _edge_stats(e, tile=8000):
    E, F = e.shape
    return pl.pallas_call(
        _estats_body,
        grid=(E // tile,),
        in_specs=[pl.BlockSpec((tile, F), lambda i: (i, 0))],
        out_specs=[pl.BlockSpec((1, F), lambda i: (0, 0))] * 2,
        out_shape=[jax.ShapeDtypeStruct((1, F), jnp.float32)] * 2,
    )(e)


def _node1_body(x_ref, g_ref, b_ref, root_ref, bias_ref, xn_ref, r1_ref):
    x = x_ref[...]
    m = jnp.mean(x, axis=0, keepdims=True)
    var = jnp.mean(x * x, axis=0, keepdims=True) - m * m
    xn = (x - m) * (g_ref[...] * lax.rsqrt(var + 1e-5)) + b_ref[...]
    xn_ref[...] = xn
    r1_ref[...] = (
        jnp.dot(xn, root_ref[...], preferred_element_type=jnp.float32)
        + bias_ref[...]
    )


def _node1(x, g, b, root, bias):
    N, F = x.shape
    Fo = root.shape[1]
    return pl.pallas_call(
        _node1_body,
        out_shape=[
            jax.ShapeDtypeStruct((N, F), jnp.float32),
            jax.ShapeDtypeStruct((N, Fo), jnp.float32),
        ],
    )(x, g.reshape(1, -1), b.reshape(1, -1), root, bias.reshape(1, -1))


def _node2_body(parts_ref, root_ref, bias_ref, h_ref, r_ref):
    h = parts_ref[0] + parts_ref[1]
    h_ref[...] = h
    r_ref[...] = (
        jnp.dot(h, root_ref[...], preferred_element_type=jnp.float32)
        + bias_ref[...]
    )


def _node2(parts, root, bias):
    _, N, F = parts.shape
    Fo = root.shape[1]
    return pl.pallas_call(
        _node2_body,
        out_shape=[
            jax.ShapeDtypeStruct((N, F), jnp.float32),
            jax.ShapeDtypeStruct((N, Fo), jnp.float32),
        ],
    )(parts, root, bias.reshape(1, -1))


def _hsum_body(parts_ref, h_ref):
    h_ref[...] = parts_ref[0] + parts_ref[1]


def _hsum(parts):
    _, N, F = parts.shape
    return pl.pallas_call(
        _hsum_body,
        out_shape=jax.ShapeDtypeStruct((N, F), jnp.float32),
    )(parts)


def _make_msg_body(E, fan_in, fan_out):
    def body(e_ref, xs_ref, eg_ref, eb_ref, s_ref, q_ref,
             w0_ref, b0_ref, w1_ref, b1_ref, out_ref):
        mean = s_ref[...] / E
        var = q_ref[...] / E - mean * mean
        en = (e_ref[...] - mean) * (eg_ref[...] * lax.rsqrt(var + 1e-5)) + eb_ref[...]
        u = _leaky(jnp.dot(en, w0_ref[...], preferred_element_type=jnp.float32)
                   + b0_ref[...])
        w = _leaky(jnp.dot(u, w1_ref[...], preferred_element_type=jnp.float32)
                   + b1_ref[...])
        xs = xs_ref[...]
        acc = xs[:, 0:1] * w[:, 0:fan_out]
        for i in range(1, fan_in):
            acc = acc + xs[:, i:i + 1] * w[:, i * fan_out:(i + 1) * fan_out]
        out_ref[...] = acc
    return body


def _msg(e, xs, eg, eb, es, eq, w0, b0, w1, b1, fan_in, fan_out, tile=1000):
    E, F = e.shape
    fhid = w0.shape[1]
    body = _make_msg_body(E, fan_in, fan_out)
    wide = w1.shape[1]
    return pl.pallas_call(
        body,
        grid=(E // tile,),
        in_specs=[
            pl.BlockSpec((tile, F), lambda i: (i, 0)),
            pl.BlockSpec((tile, fan_in), lambda i: (i, 0)),
            pl.BlockSpec((1, F), lambda i: (0, 0)),
            pl.BlockSpec((1, F), lambda i: (0, 0)),
            pl.BlockSpec((1, F), lambda i: (0, 0)),
            pl.BlockSpec((1, F), lambda i: (0, 0)),
            pl.BlockSpec((F, fhid), lambda i: (0, 0)),
            pl.BlockSpec((1, fhid), lambda i: (0, 0)),
            pl.BlockSpec((fhid, wide), lambda i: (0, 0)),
            pl.BlockSpec((1, wide), lambda i: (0, 0)),
        ],
        out_specs=pl.BlockSpec((tile, fan_out), lambda i: (i, 0)),
        out_shape=jax.ShapeDtypeStruct((E, fan_out), jnp.float32),
    )(e, xs, eg.reshape(1, -1), eb.reshape(1, -1), es, eq,
      w0, b0.reshape(1, -1), w1, b1.reshape(1, -1))


def _make_final_body(E):
    def body(e_ref, hs_ref, hd_ref, eg_ref, eb_ref, s_ref, q_ref,
             w0a_ref, w0b_ref, w0c_ref, b0_ref, w1_ref, b1_ref,
             w2_ref, b2_ref, w3_ref, b3_ref, w4_ref, b4_ref, out_ref):
        mean = s_ref[...] / E
        var = q_ref[...] / E - mean * mean
        en = (e_ref[...] - mean) * (eg_ref[...] * lax.rsqrt(var + 1e-5)) + eb_ref[...]
        t = _leaky(
            jnp.dot(hs_ref[...], w0a_ref[...], preferred_element_type=jnp.float32)
            + jnp.dot(hd_ref[...], w0b_ref[...], preferred_element_type=jnp.float32)
            + jnp.dot(en, w0c_ref[...], preferred_element_type=jnp.float32)
            + b0_ref[...])
        t = _leaky(jnp.dot(t, w1_ref[...], preferred_element_type=jnp.float32) + b1_ref[...])
        t = _leaky(jnp.dot(t, w2_ref[...], preferred_element_type=jnp.float32) + b2_ref[...])
        t = _leaky(jnp.dot(t, w3_ref[...], preferred_element_type=jnp.float32) + b3_ref[...])
        out_ref[...] = jnp.dot(t, w4_ref[...], preferred_element_type=jnp.float32) + b4_ref[...]
    return body


def _final_mlp(e, hs, hd, eg, eb, es, eq, w0, b0, w1, b1, w2, b2, w3, b3,
               w4, b4, tile=1000):
    E, F = e.shape
    H = hs.shape[1]
    w0a, w0b, w0c = w0[:H], w0[H:2 * H], w0[2 * H:]
    full = lambda a: pl.BlockSpec(a.shape, lambda i: tuple(0 for _ in a.shape))
    b0r, b1r, b2r, b3r, b4r = (v.reshape(1, -1) for v in (b0, b1, b2, b3, b4))
    args = (e, hs, hd, eg.reshape(1, -1), eb.reshape(1, -1), es, eq,
            w0a, w0b, w0c, b0r, w1, b1r, w2, b2r, w3, b3r, w4, b4r)
    in_specs = [
        pl.BlockSpec((tile, F), lambda i: (i, 0)),
        pl.BlockSpec((tile, H), lambda i: (i, 0)),
        pl.BlockSpec((tile, H), lambda i: (i, 0)),
    ] + [full(a) for a in args[3:]]
    return pl.pallas_call(
        _make_final_body(E),
        grid=(E // tile,),
        in_specs=in_specs,
        out_specs=pl.BlockSpec((tile, 2), lambda i: (i, 0)),
        out_shape=jax.ShapeDtypeStruct((E, 2), jnp.float32),
    )(*args)


def _sc_gather(table, *idxs):
    if _XLA_GATHER:
        return tuple(jnp.take(table, i, axis=0) for i in idxs)
    Nn, D = table.shape
    E = idxs[0].shape[0]
    per_w = E // _NW
    n_full, rem = divmod(per_w, _CH)
    n_idx = len(idxs)
    mesh = plsc.VectorSubcoreMesh(core_axis_name="c", subcore_axis_name="s",
                                  num_cores=_NC, num_subcores=_NS)
    scratch = [
        pltpu.VMEM((_CH,), jnp.int32),
        pltpu.VMEM((_CH, D), jnp.float32),
        pltpu.VMEM((max(rem, 1),), jnp.int32),
        pltpu.VMEM((max(rem, 1), D), jnp.float32),
        pltpu.SemaphoreType.DMA,
    ]

    def body(table_ref, *rest):
        idx_refs = rest[:n_idx]
        out_refs = rest[n_idx:2 * n_idx]
        idx_c, buf, idx_r, buf_r, sem = rest[2 * n_idx:]
        wid = lax.axis_index("s") * _NC + lax.axis_index("c")
        base = wid * per_w
        for k in range(n_idx):
            def chunk(j, _, k=k):
                off = base + j * _CH
                pltpu.sync_copy(idx_refs[k].at[pl.ds(off, _CH)], idx_c)
                pltpu.async_copy(table_ref.at[idx_c], buf, sem).wait()
                pltpu.sync_copy(buf, out_refs[k].at[pl.ds(off, _CH)])
                return 0
            lax.fori_loop(0, n_full, chunk, 0, unroll=False)
            if rem:
                off = base + n_full * _CH
                pltpu.sync_copy(idx_refs[k].at[pl.ds(off, rem)], idx_r)
                pltpu.async_copy(table_ref.at[idx_r], buf_r, sem).wait()
                pltpu.sync_copy(buf_r, out_refs[k].at[pl.ds(off, rem)])

    out_type = tuple(jax.ShapeDtypeStruct((E, D), jnp.float32)
                     for _ in range(n_idx))
    fn = pl.kernel(body, out_type=out_type, mesh=mesh, scratch_types=scratch,
                   compiler_params=pltpu.CompilerParams(use_tc_tiling_on_sc=False))
    return fn(table, *idxs)


def _sc_scatter(msgs, dst, seeds):
    E, D = msgs.shape
    Nn = seeds.shape[1]
    per_w = E // _NW
    n_full, rem = divmod(per_w, _CH)
    rpt = (Nn // _NS) // 8 * 8
    tail = Nn - rpt * _NS
    mesh = plsc.VectorSubcoreMesh(core_axis_name="c", subcore_axis_name="s",
                                  num_cores=_NC, num_subcores=_NS)
    scratch = [
        pltpu.VMEM((1, _CH), jnp.int32),
        pltpu.VMEM((_CH, D), jnp.float32),
        pltpu.VMEM((1, max(rem, 1)), jnp.int32),
        pltpu.VMEM((max(rem, 1), D), jnp.float32),
        pltpu.VMEM_SHARED((Nn, D), jnp.float32),
    ]

    def body(m_ref, dst_ref, seeds_ref, out_ref, idx_c, buf, idx_r, buf_r, acc):
        cid = lax.axis_index("c")
        sid = lax.axis_index("s")
        wid = sid * _NC + cid
        base = wid * per_w
        pltpu.sync_copy(seeds_ref.at[cid, pl.ds(sid * rpt, rpt)],
                        acc.at[pl.ds(sid * rpt, rpt)])
        if tail:
            @pl.when(sid == 0)
            def _tail_init():
                pltpu.sync_copy(seeds_ref.at[cid, pl.ds(rpt * _NS, tail)],
                                acc.at[pl.ds(rpt * _NS, tail)])
        plsc.subcore_barrier()

        def chunk(j, _):
            off = base + j * _CH
            pltpu.sync_copy(dst_ref.at[pl.ds(off, _CH)], idx_c.at[0])
            pltpu.sync_copy(m_ref.at[pl.ds(off, _CH)], buf)
            pltpu.sync_copy(buf, acc.at[idx_c.at[0]], add=True)
            return 0
        lax.fori_loop(0, n_full, chunk, 0, unroll=False)
        if rem:
            off = base + n_full * _CH
            pltpu.sync_copy(dst_ref.at[pl.ds(off, rem)], idx_r.at[0])
            pltpu.sync_copy(m_ref.at[pl.ds(off, rem)], buf_r)
            pltpu.sync_copy(buf_r, acc.at[idx_r.at[0]], add=True)
        plsc.subcore_barrier()
        pltpu.sync_copy(acc.at[pl.ds(sid * rpt, rpt)],
                        out_ref.at[cid, pl.ds(sid * rpt, rpt)])
        if tail:
            @pl.when(sid == 0)
            def _tail_out():
                pltpu.sync_copy(acc.at[pl.ds(rpt * _NS, tail)],
                                out_ref.at[cid, pl.ds(rpt * _NS, tail)])

    fn = pl.kernel(body,
                   out_type=jax.ShapeDtypeStruct((_NC, Nn, D), jnp.float32),
                   mesh=mesh, scratch_types=scratch,
                   compiler_params=pltpu.CompilerParams(use_tc_tiling_on_sc=False))
    return fn(msgs, dst, seeds)


def kernel(x, edge_index, e, xbatch,
           bn_node_g, bn_node_b, bn_edge_g, bn_edge_b,
           nn1_W0, nn1_b0, nn1_W1, nn1_b1, conv1_root, conv1_bias,
           nn2_W0, nn2_b0, nn2_W1, nn2_b1, conv2_root, conv2_bias,
           mlp_W0, mlp_b0, mlp_W1, mlp_b1, mlp_W2, mlp_b2,
           mlp_W3, mlp_b3, mlp_W4, mlp_b4):
    src = edge_index[0]
    dst = edge_index[1]

    es, eq = _edge_stats(e)
    xn, r1 = _node1(x, bn_node_g, bn_node_b, conv1_root, conv1_bias)

    (xs,) = _sc_gather(xn, src)
    m1 = _msg(e, xs, bn_edge_g, bn_edge_b, es, eq,
              nn1_W0, nn1_b0, nn1_W1, nn1_b1, fan_in=16, fan_out=32)
    seeds1 = jnp.stack([r1, jnp.zeros_like(r1)])
    if _XLA_SCATTER:
        parts1 = jnp.stack([r1 + jnp.zeros_like(r1).at[dst].add(m1),
                            jnp.zeros_like(r1)])
    else:
        parts1 = _sc_scatter(m1, dst, seeds1)
    h1, r2 = _node2(parts1, conv2_root, conv2_bias)

    (h1s,) = _sc_gather(h1, src)
    m2 = _msg(e, h1s, bn_edge_g, bn_edge_b, es, eq,
              nn2_W0, nn2_b0, nn2_W1, nn2_b1, fan_in=32, fan_out=64)
    seeds2 = jnp.stack([r2, jnp.zeros_like(r2)])
    if _XLA_SCATTER:
        parts2 = jnp.stack([r2 + jnp.zeros_like(r2).at[dst].add(m2),
                            jnp.zeros_like(r2)])
    else:
        parts2 = _sc_scatter(m2, dst, seeds2)
    h2 = _hsum(parts2)

    h2s, h2d = _sc_gather(h2, src, dst)
    return _final_mlp(e, h2s, h2d, bn_edge_g, bn_edge_b, es, eq,
                      mlp_W0, mlp_b0, mlp_W1, mlp_b1, mlp_W2, mlp_b2,
                      mlp_W3, mlp_b3, mlp_W4, mlp_b4)


# speedup vs baseline: 1.1837x; 1.0853x over previous
import functools

import jax
import jax.numpy as jnp
from jax import lax
from jax.experimental import pallas as pl
from jax.experimental.pallas import tpu as pltpu
from jax.experimental.pallas import tpu_sc as plsc

_NC, _NS = 2, 16
_NW = _NC * _NS
_CH = 128


_XLA_SCATTER = False
_XLA_GATHER = False


def _dot(a, b):
    return jnp.dot(a.astype(jnp.bfloat16), b.astype(jnp.bfloat16),
                   preferred_element_type=jnp.float32)

def _leaky(v):
    return jnp.where(v >= 0, v, 0.1 * v)


def _estats_body(e_ref, s_ref, q_ref):
    i = pl.program_id(0)

    @pl.when(i == 0)
    def _init():
        s_ref[...] = jnp.zeros_like(s_ref)
        q_ref[...] = jnp.zeros_like(q_ref)

    blk = e_ref[...]
    s_ref[...] += jnp.sum(blk, axis=0, keepdims=True)
    q_ref[...] += jnp.sum(blk * blk, axis=0, keepdims=True)


def _edge_stats(e, tile=8000):
    E, F = e.shape
    return pl.pallas_call(
        _estats_body,
        grid=(E // tile,),
        in_specs=[pl.BlockSpec((tile, F), lambda i: (i, 0))],
        out_specs=[pl.BlockSpec((1, F), lambda i: (0, 0))] * 2,
        out_shape=[jax.ShapeDtypeStruct((1, F), jnp.float32)] * 2,
    )(e)


def _node1_body(x_ref, g_ref, b_ref, root_ref, bias_ref, xn_ref, r1_ref):
    x = x_ref[...]
    m = jnp.mean(x, axis=0, keepdims=True)
    var = jnp.mean(x * x, axis=0, keepdims=True) - m * m
    xn = (x - m) * (g_ref[...] * lax.rsqrt(var + 1e-5)) + b_ref[...]
    xn_ref[...] = xn
    r1_ref[...] = (
        _dot(xn, root_ref[...])
        + bias_ref[...]
    )


def _node1(x, g, b, root, bias):
    N, F = x.shape
    Fo = root.shape[1]
    return pl.pallas_call(
        _node1_body,
        out_shape=[
            jax.ShapeDtypeStruct((N, F), jnp.float32),
            jax.ShapeDtypeStruct((N, Fo), jnp.float32),
        ],
    )(x, g.reshape(1, -1), b.reshape(1, -1), root, bias.reshape(1, -1))


def _node2_body(parts_ref, root_ref, bias_ref, h_ref, r_ref):
    h = parts_ref[0] + parts_ref[1]
    h_ref[...] = h
    r_ref[...] = (
        _dot(h, root_ref[...])
        + bias_ref[...]
    )


def _node2(parts, root, bias):
    _, N, F = parts.shape
    Fo = root.shape[1]
    return pl.pallas_call(
        _node2_body,
        out_shape=[
            jax.ShapeDtypeStruct((N, F), jnp.float32),
            jax.ShapeDtypeStruct((N, Fo), jnp.float32),
        ],
    )(parts, root, bias.reshape(1, -1))


def _hsum_body(parts_ref, h_ref):
    h_ref[...] = parts_ref[0] + parts_ref[1]


def _hsum(parts):
    _, N, F = parts.shape
    return pl.pallas_call(
        _hsum_body,
        out_shape=jax.ShapeDtypeStruct((N, F), jnp.float32),
    )(parts)


def _make_msg_body(E, fan_in, fan_out):
    def body(e_ref, xs_ref, sel_ref, eg_ref, eb_ref, s_ref, q_ref,
             w0_ref, b0_ref, w1_ref, b1_ref, out_ref):
        mean = s_ref[...] / E
        var = q_ref[...] / E - mean * mean
        en = (e_ref[...] - mean) * (eg_ref[...] * lax.rsqrt(var + 1e-5)) + eb_ref[...]
        u = _leaky(_dot(en, w0_ref[...]) + b0_ref[...])
        w = _leaky(_dot(u, w1_ref[...]) + b1_ref[...])
        w = w.astype(jnp.bfloat16).astype(jnp.float32)
        xs = xs_ref[...].astype(jnp.bfloat16).astype(jnp.float32)
        xs_rep = jnp.dot(xs, sel_ref[...],
                         preferred_element_type=jnp.float32,
                         precision=jax.lax.Precision.HIGHEST)
        acc = xs_rep * w
        while acc.shape[1] > fan_out:
            half = acc.shape[1] // 2
            acc = acc[:, :half] + acc[:, half:]
        out_ref[...] = acc
    return body


def _msg(e, xs, eg, eb, es, eq, w0, b0, w1, b1, fan_in, fan_out, tile=1000):
    E, F = e.shape
    fhid = w0.shape[1]
    body = _make_msg_body(E, fan_in, fan_out)
    wide = w1.shape[1]
    return pl.pallas_call(
        body,
        grid=(E // tile,),
        in_specs=[
            pl.BlockSpec((tile, F), lambda i: (i, 0)),
            pl.BlockSpec((tile, fan_in), lambda i: (i, 0)),
            pl.BlockSpec((fan_in, wide), lambda i: (0, 0)),
            pl.BlockSpec((1, F), lambda i: (0, 0)),
            pl.BlockSpec((1, F), lambda i: (0, 0)),
            pl.BlockSpec((1, F), lambda i: (0, 0)),
            pl.BlockSpec((1, F), lambda i: (0, 0)),
            pl.BlockSpec((F, fhid), lambda i: (0, 0)),
            pl.BlockSpec((1, fhid), lambda i: (0, 0)),
            pl.BlockSpec((fhid, wide), lambda i: (0, 0)),
            pl.BlockSpec((1, wide), lambda i: (0, 0)),
        ],
        out_specs=pl.BlockSpec((tile, fan_out), lambda i: (i, 0)),
        out_shape=jax.ShapeDtypeStruct((E, fan_out), jnp.float32),
    )(e, xs, jnp.repeat(jnp.eye(fan_in, dtype=jnp.float32), fan_out, axis=1),
      eg.reshape(1, -1), eb.reshape(1, -1), es, eq,
      w0, b0.reshape(1, -1), w1, b1.reshape(1, -1))


def _make_final_body(E):
    def body(e_ref, hs_ref, hd_ref, eg_ref, eb_ref, s_ref, q_ref,
             w0a_ref, w0b_ref, w0c_ref, b0_ref, w1_ref, b1_ref,
             w2_ref, b2_ref, w3_ref, b3_ref, w4_ref, b4_ref, out_ref):
        mean = s_ref[...] / E
        var = q_ref[...] / E - mean * mean
        en = (e_ref[...] - mean) * (eg_ref[...] * lax.rsqrt(var + 1e-5)) + eb_ref[...]
        t = _leaky(_dot(hs_ref[...], w0a_ref[...])
                   + _dot(hd_ref[...], w0b_ref[...])
                   + _dot(en, w0c_ref[...])
                   + b0_ref[...])
        t = _leaky(_dot(t, w1_ref[...]) + b1_ref[...])
        t = _leaky(_dot(t, w2_ref[...]) + b2_ref[...])
        t = _leaky(_dot(t, w3_ref[...]) + b3_ref[...])
        out_ref[...] = _dot(t, w4_ref[...]) + b4_ref[...]
    return body


def _final_mlp(e, hs, hd, eg, eb, es, eq, w0, b0, w1, b1, w2, b2, w3, b3,
               w4, b4, tile=1000):
    E, F = e.shape
    H = hs.shape[1]
    w0a, w0b, w0c = w0[:H], w0[H:2 * H], w0[2 * H:]
    full = lambda a: pl.BlockSpec(a.shape, lambda i: tuple(0 for _ in a.shape))
    b0r, b1r, b2r, b3r, b4r = (v.reshape(1, -1) for v in (b0, b1, b2, b3, b4))
    args = (e, hs, hd, eg.reshape(1, -1), eb.reshape(1, -1), es, eq,
            w0a, w0b, w0c, b0r, w1, b1r, w2, b2r, w3, b3r, w4, b4r)
    in_specs = [
        pl.BlockSpec((tile, F), lambda i: (i, 0)),
        pl.BlockSpec((tile, H), lambda i: (i, 0)),
        pl.BlockSpec((tile, H), lambda i: (i, 0)),
    ] + [full(a) for a in args[3:]]
    return pl.pallas_call(
        _make_final_body(E),
        grid=(E // tile,),
        in_specs=in_specs,
        out_specs=pl.BlockSpec((tile, 2), lambda i: (i, 0)),
        out_shape=jax.ShapeDtypeStruct((E, 2), jnp.float32),
    )(*args)


def _sc_gather(table, *idxs):
    if _XLA_GATHER:
        return tuple(jnp.take(table, i, axis=0) for i in idxs)
    Nn, D = table.shape
    E = idxs[0].shape[0]
    per_w = E // _NW
    n_full, rem = divmod(per_w, _CH)
    n_idx = len(idxs)
    mesh = plsc.VectorSubcoreMesh(core_axis_name="c", subcore_axis_name="s",
                                  num_cores=_NC, num_subcores=_NS)
    scratch = [
        pltpu.VMEM((_CH,), jnp.int32),
        pltpu.VMEM((_CH, D), jnp.float32),
        pltpu.VMEM((max(rem, 1),), jnp.int32),
        pltpu.VMEM((max(rem, 1), D), jnp.float32),
        pltpu.SemaphoreType.DMA,
    ]

    def body(table_ref, *rest):
        idx_refs = rest[:n_idx]
        out_refs = rest[n_idx:2 * n_idx]
        idx_c, buf, idx_r, buf_r, sem = rest[2 * n_idx:]
        wid = lax.axis_index("s") * _NC + lax.axis_index("c")
        base = wid * per_w
        for k in range(n_idx):
            def chunk(j, _, k=k):
                off = base + j * _CH
                pltpu.sync_copy(idx_refs[k].at[pl.ds(off, _CH)], idx_c)
                pltpu.async_copy(table_ref.at[idx_c], buf, sem).wait()
                pltpu.sync_copy(buf, out_refs[k].at[pl.ds(off, _CH)])
                return 0
            lax.fori_loop(0, n_full, chunk, 0, unroll=False)
            if rem:
                off = base + n_full * _CH
                pltpu.sync_copy(idx_refs[k].at[pl.ds(off, rem)], idx_r)
                pltpu.async_copy(table_ref.at[idx_r], buf_r, sem).wait()
                pltpu.sync_copy(buf_r, out_refs[k].at[pl.ds(off, rem)])

    out_type = tuple(jax.ShapeDtypeStruct((E, D), jnp.float32)
                     for _ in range(n_idx))
    fn = pl.kernel(body, out_type=out_type, mesh=mesh, scratch_types=scratch,
                   compiler_params=pltpu.CompilerParams(use_tc_tiling_on_sc=False))
    return fn(table, *idxs)


def _sc_scatter(msgs, dst, seeds):
    E, D = msgs.shape
    Nn = seeds.shape[1]
    per_w = E // _NW
    n_full, rem = divmod(per_w, _CH)
    rpt = (Nn // _NS) // 8 * 8
    tail = Nn - rpt * _NS
    mesh = plsc.VectorSubcoreMesh(core_axis_name="c", subcore_axis_name="s",
                                  num_cores=_NC, num_subcores=_NS)
    scratch = [
        pltpu.VMEM((1, _CH), jnp.int32),
        pltpu.VMEM((_CH, D), jnp.float32),
        pltpu.VMEM((1, max(rem, 1)), jnp.int32),
        pltpu.VMEM((max(rem, 1), D), jnp.float32),
        pltpu.VMEM_SHARED((Nn, D), jnp.float32),
    ]

    def body(m_ref, dst_ref, seeds_ref, out_ref, idx_c, buf, idx_r, buf_r, acc):
        cid = lax.axis_index("c")
        sid = lax.axis_index("s")
        wid = sid * _NC + cid
        base = wid * per_w
        pltpu.sync_copy(seeds_ref.at[cid, pl.ds(sid * rpt, rpt)],
                        acc.at[pl.ds(sid * rpt, rpt)])
        if tail:
            @pl.when(sid == 0)
            def _tail_init():
                pltpu.sync_copy(seeds_ref.at[cid, pl.ds(rpt * _NS, tail)],
                                acc.at[pl.ds(rpt * _NS, tail)])
        plsc.subcore_barrier()

        def chunk(j, _):
            off = base + j * _CH
            pltpu.sync_copy(dst_ref.at[pl.ds(off, _CH)], idx_c.at[0])
            pltpu.sync_copy(m_ref.at[pl.ds(off, _CH)], buf)
            pltpu.sync_copy(buf, acc.at[idx_c.at[0]], add=True)
            return 0
        lax.fori_loop(0, n_full, chunk, 0, unroll=False)
        if rem:
            off = base + n_full * _CH
            pltpu.sync_copy(dst_ref.at[pl.ds(off, rem)], idx_r.at[0])
            pltpu.sync_copy(m_ref.at[pl.ds(off, rem)], buf_r)
            pltpu.sync_copy(buf_r, acc.at[idx_r.at[0]], add=True)
        plsc.subcore_barrier()
        pltpu.sync_copy(acc.at[pl.ds(sid * rpt, rpt)],
                        out_ref.at[cid, pl.ds(sid * rpt, rpt)])
        if tail:
            @pl.when(sid == 0)
            def _tail_out():
                pltpu.sync_copy(acc.at[pl.ds(rpt * _NS, tail)],
                                out_ref.at[cid, pl.ds(rpt * _NS, tail)])

    fn = pl.kernel(body,
                   out_type=jax.ShapeDtypeStruct((_NC, Nn, D), jnp.float32),
                   mesh=mesh, scratch_types=scratch,
                   compiler_params=pltpu.CompilerParams(use_tc_tiling_on_sc=False))
    return fn(msgs, dst, seeds)


def kernel(x, edge_index, e, xbatch,
           bn_node_g, bn_node_b, bn_edge_g, bn_edge_b,
           nn1_W0, nn1_b0, nn1_W1, nn1_b1, conv1_root, conv1_bias,
           nn2_W0, nn2_b0, nn2_W1, nn2_b1, conv2_root, conv2_bias,
           mlp_W0, mlp_b0, mlp_W1, mlp_b1, mlp_W2, mlp_b2,
           mlp_W3, mlp_b3, mlp_W4, mlp_b4):
    src = edge_index[0]
    dst = edge_index[1]

    es, eq = _edge_stats(e)
    xn, r1 = _node1(x, bn_node_g, bn_node_b, conv1_root, conv1_bias)

    (xs,) = _sc_gather(xn, src)
    m1 = _msg(e, xs, bn_edge_g, bn_edge_b, es, eq,
              nn1_W0, nn1_b0, nn1_W1, nn1_b1, fan_in=16, fan_out=32)
    seeds1 = jnp.stack([r1, jnp.zeros_like(r1)])
    if _XLA_SCATTER:
        parts1 = jnp.stack([r1 + jnp.zeros_like(r1).at[dst].add(m1),
                            jnp.zeros_like(r1)])
    else:
        parts1 = _sc_scatter(m1, dst, seeds1)
    h1, r2 = _node2(parts1, conv2_root, conv2_bias)

    (h1s,) = _sc_gather(h1, src)
    m2 = _msg(e, h1s, bn_edge_g, bn_edge_b, es, eq,
              nn2_W0, nn2_b0, nn2_W1, nn2_b1, fan_in=32, fan_out=64)
    seeds2 = jnp.stack([r2, jnp.zeros_like(r2)])
    if _XLA_SCATTER:
        parts2 = jnp.stack([r2 + jnp.zeros_like(r2).at[dst].add(m2),
                            jnp.zeros_like(r2)])
    else:
        parts2 = _sc_scatter(m2, dst, seeds2)
    h2 = _hsum(parts2)

    h2s, h2d = _sc_gather(h2, src, dst)
    return _final_mlp(e, h2s, h2d, bn_edge_g, bn_edge_b, es, eq,
                      mlp_W0, mlp_b0, mlp_W1, mlp_b1, mlp_W2, mlp_b2,
                      mlp_W3, mlp_b3, mlp_W4, mlp_b4)


# speedup vs baseline: 1.7780x; 1.5020x over previous
import functools

import jax
import jax.numpy as jnp
from jax import lax
from jax.experimental import pallas as pl
from jax.experimental.pallas import tpu as pltpu
from jax.experimental.pallas import tpu_sc as plsc

_NC, _NS = 2, 16
_NW = _NC * _NS
_CH = 128


_XLA_SCATTER = False
_XLA_GATHER = False


def _dot(a, b):
    return jnp.dot(a.astype(jnp.bfloat16), b.astype(jnp.bfloat16),
                   preferred_element_type=jnp.float32)

def _leaky(v):
    return jnp.where(v >= 0, v, 0.1 * v)


def _estats_body(e_ref, s_ref, q_ref):
    i = pl.program_id(0)

    @pl.when(i == 0)
    def _init():
        s_ref[...] = jnp.zeros_like(s_ref)
        q_ref[...] = jnp.zeros_like(q_ref)

    blk = e_ref[...]
    s_ref[...] += jnp.sum(blk, axis=0, keepdims=True)
    q_ref[...] += jnp.sum(blk * blk, axis=0, keepdims=True)


def _edge_stats(e, tile=8000):
    E, F = e.shape
    return pl.pallas_call(
        _estats_body,
        grid=(E // tile,),
        in_specs=[pl.BlockSpec((tile, F), lambda i: (i, 0))],
        out_specs=[pl.BlockSpec((1, F), lambda i: (0, 0))] * 2,
        out_shape=[jax.ShapeDtypeStruct((1, F), jnp.float32)] * 2,
    )(e)


def _node1_body(x_ref, g_ref, b_ref, root_ref, bias_ref, xn_ref, r1_ref):
    x = x_ref[...]
    m = jnp.mean(x, axis=0, keepdims=True)
    var = jnp.mean(x * x, axis=0, keepdims=True) - m * m
    xn = (x - m) * (g_ref[...] * lax.rsqrt(var + 1e-5)) + b_ref[...]
    xn_ref[...] = xn
    r1_ref[...] = (
        _dot(xn, root_ref[...])
        + bias_ref[...]
    )


def _node1(x, g, b, root, bias):
    N, F = x.shape
    Fo = root.shape[1]
    return pl.pallas_call(
        _node1_body,
        out_shape=[
            jax.ShapeDtypeStruct((N, F), jnp.float32),
            jax.ShapeDtypeStruct((N, Fo), jnp.float32),
        ],
    )(x, g.reshape(1, -1), b.reshape(1, -1), root, bias.reshape(1, -1))


def _node2_body(parts_ref, root_ref, bias_ref, h_ref, r_ref):
    h = parts_ref[0] + parts_ref[1]
    h_ref[...] = h
    r_ref[...] = (
        _dot(h, root_ref[...])
        + bias_ref[...]
    )


def _node2(parts, root, bias):
    _, N, F = parts.shape
    Fo = root.shape[1]
    return pl.pallas_call(
        _node2_body,
        out_shape=[
            jax.ShapeDtypeStruct((N, F), jnp.float32),
            jax.ShapeDtypeStruct((N, Fo), jnp.float32),
        ],
    )(parts, root, bias.reshape(1, -1))


def _hsum_body(parts_ref, h_ref):
    h_ref[...] = parts_ref[0] + parts_ref[1]


def _hsum(parts):
    _, N, F = parts.shape
    return pl.pallas_call(
        _hsum_body,
        out_shape=jax.ShapeDtypeStruct((N, F), jnp.float32),
    )(parts)


def _make_msg_body(E, fan_in, fan_out):
    def body(e_ref, xs_ref, sel_ref, eg_ref, eb_ref, s_ref, q_ref,
             w0_ref, b0_ref, w1_ref, b1_ref, out_ref):
        mean = s_ref[...] / E
        var = q_ref[...] / E - mean * mean
        en = (e_ref[...] - mean) * (eg_ref[...] * lax.rsqrt(var + 1e-5)) + eb_ref[...]
        u = _leaky(_dot(en, w0_ref[...]) + b0_ref[...])
        w = _leaky(_dot(u, w1_ref[...]) + b1_ref[...])
        w = w.astype(jnp.bfloat16).astype(jnp.float32)
        xs = xs_ref[...].astype(jnp.bfloat16).astype(jnp.float32)
        xs_rep = _dot(xs, sel_ref[...])
        acc = xs_rep * w
        while acc.shape[1] > fan_out:
            half = acc.shape[1] // 2
            acc = acc[:, :half] + acc[:, half:]
        out_ref[...] = acc
    return body


def _msg(e, xs, eg, eb, es, eq, w0, b0, w1, b1, fan_in, fan_out, tile=1000):
    E, F = e.shape
    fhid = w0.shape[1]
    body = _make_msg_body(E, fan_in, fan_out)
    wide = w1.shape[1]
    return pl.pallas_call(
        body,
        grid=(E // tile,),
        in_specs=[
            pl.BlockSpec((tile, F), lambda i: (i, 0)),
            pl.BlockSpec((tile, fan_in), lambda i: (i, 0)),
            pl.BlockSpec((fan_in, wide), lambda i: (0, 0)),
            pl.BlockSpec((1, F), lambda i: (0, 0)),
            pl.BlockSpec((1, F), lambda i: (0, 0)),
            pl.BlockSpec((1, F), lambda i: (0, 0)),
            pl.BlockSpec((1, F), lambda i: (0, 0)),
            pl.BlockSpec((F, fhid), lambda i: (0, 0)),
            pl.BlockSpec((1, fhid), lambda i: (0, 0)),
            pl.BlockSpec((fhid, wide), lambda i: (0, 0)),
            pl.BlockSpec((1, wide), lambda i: (0, 0)),
        ],
        out_specs=pl.BlockSpec((tile, fan_out), lambda i: (i, 0)),
        out_shape=jax.ShapeDtypeStruct((E, fan_out), jnp.float32),
    )(e, xs, jnp.repeat(jnp.eye(fan_in, dtype=jnp.float32), fan_out, axis=1),
      eg.reshape(1, -1), eb.reshape(1, -1), es, eq,
      w0, b0.reshape(1, -1), w1, b1.reshape(1, -1))


def _make_final_body(E):
    def body(e_ref, hs_ref, hd_ref, eg_ref, eb_ref, s_ref, q_ref,
             w0a_ref, w0b_ref, w0c_ref, b0_ref, w1_ref, b1_ref,
             w2_ref, b2_ref, w3_ref, b3_ref, w4_ref, b4_ref, out_ref):
        mean = s_ref[...] / E
        var = q_ref[...] / E - mean * mean
        en = (e_ref[...] - mean) * (eg_ref[...] * lax.rsqrt(var + 1e-5)) + eb_ref[...]
        t = _leaky(_dot(hs_ref[...], w0a_ref[...])
                   + _dot(hd_ref[...], w0b_ref[...])
                   + _dot(en, w0c_ref[...])
                   + b0_ref[...])
        t = _leaky(_dot(t, w1_ref[...]) + b1_ref[...])
        t = _leaky(_dot(t, w2_ref[...]) + b2_ref[...])
        t = _leaky(_dot(t, w3_ref[...]) + b3_ref[...])
        out_ref[...] = _dot(t, w4_ref[...]) + b4_ref[...]
    return body


def _final_mlp(e, hs, hd, eg, eb, es, eq, w0, b0, w1, b1, w2, b2, w3, b3,
               w4, b4, tile=1000):
    E, F = e.shape
    H = hs.shape[1]
    w0a, w0b, w0c = w0[:H], w0[H:2 * H], w0[2 * H:]
    full = lambda a: pl.BlockSpec(a.shape, lambda i: tuple(0 for _ in a.shape))
    b0r, b1r, b2r, b3r, b4r = (v.reshape(1, -1) for v in (b0, b1, b2, b3, b4))
    args = (e, hs, hd, eg.reshape(1, -1), eb.reshape(1, -1), es, eq,
            w0a, w0b, w0c, b0r, w1, b1r, w2, b2r, w3, b3r, w4, b4r)
    in_specs = [
        pl.BlockSpec((tile, F), lambda i: (i, 0)),
        pl.BlockSpec((tile, H), lambda i: (i, 0)),
        pl.BlockSpec((tile, H), lambda i: (i, 0)),
    ] + [full(a) for a in args[3:]]
    return pl.pallas_call(
        _make_final_body(E),
        grid=(E // tile,),
        in_specs=in_specs,
        out_specs=pl.BlockSpec((tile, 2), lambda i: (i, 0)),
        out_shape=jax.ShapeDtypeStruct((E, 2), jnp.float32),
    )(*args)


def _sc_gather(table, *idxs):
    if _XLA_GATHER:
        return tuple(jnp.take(table, i, axis=0) for i in idxs)
    Nn, D = table.shape
    E = idxs[0].shape[0]
    per_w = E // _NW
    n_full, rem = divmod(per_w, _CH)
    n_idx = len(idxs)
    mesh = plsc.VectorSubcoreMesh(core_axis_name="c", subcore_axis_name="s",
                                  num_cores=_NC, num_subcores=_NS)
    scratch = [
        pltpu.VMEM((_CH,), jnp.int32),
        pltpu.VMEM((_CH, D), jnp.float32),
        pltpu.VMEM((max(rem, 1),), jnp.int32),
        pltpu.VMEM((max(rem, 1), D), jnp.float32),
        pltpu.SemaphoreType.DMA,
    ]

    def body(table_ref, *rest):
        idx_refs = rest[:n_idx]
        out_refs = rest[n_idx:2 * n_idx]
        idx_c, buf, idx_r, buf_r, sem = rest[2 * n_idx:]
        wid = lax.axis_index("s") * _NC + lax.axis_index("c")
        base = wid * per_w
        for k in range(n_idx):
            def chunk(j, _, k=k):
                off = base + j * _CH
                pltpu.sync_copy(idx_refs[k].at[pl.ds(off, _CH)], idx_c)
                pltpu.async_copy(table_ref.at[idx_c], buf, sem).wait()
                pltpu.sync_copy(buf, out_refs[k].at[pl.ds(off, _CH)])
                return 0
            lax.fori_loop(0, n_full, chunk, 0, unroll=False)
            if rem:
                off = base + n_full * _CH
                pltpu.sync_copy(idx_refs[k].at[pl.ds(off, rem)], idx_r)
                pltpu.async_copy(table_ref.at[idx_r], buf_r, sem).wait()
                pltpu.sync_copy(buf_r, out_refs[k].at[pl.ds(off, rem)])

    out_type = tuple(jax.ShapeDtypeStruct((E, D), jnp.float32)
                     for _ in range(n_idx))
    fn = pl.kernel(body, out_type=out_type, mesh=mesh, scratch_types=scratch,
                   compiler_params=pltpu.CompilerParams(use_tc_tiling_on_sc=False))
    return fn(table, *idxs)


def _sc_scatter(msgs, dst, seeds):
    E, D = msgs.shape
    Nn = seeds.shape[1]
    per_w = E // _NW
    n_full, rem = divmod(per_w, _CH)
    rpt = (Nn // _NS) // 8 * 8
    tail = Nn - rpt * _NS
    mesh = plsc.VectorSubcoreMesh(core_axis_name="c", subcore_axis_name="s",
                                  num_cores=_NC, num_subcores=_NS)
    scratch = [
        pltpu.VMEM((1, _CH), jnp.int32),
        pltpu.VMEM((_CH, D), jnp.float32),
        pltpu.VMEM((1, max(rem, 1)), jnp.int32),
        pltpu.VMEM((max(rem, 1), D), jnp.float32),
        pltpu.VMEM_SHARED((Nn, D), jnp.float32),
    ]

    def body(m_ref, dst_ref, seeds_ref, out_ref, idx_c, buf, idx_r, buf_r, acc):
        cid = lax.axis_index("c")
        sid = lax.axis_index("s")
        wid = sid * _NC + cid
        base = wid * per_w
        pltpu.sync_copy(seeds_ref.at[cid, pl.ds(sid * rpt, rpt)],
                        acc.at[pl.ds(sid * rpt, rpt)])
        if tail:
            @pl.when(sid == 0)
            def _tail_init():
                pltpu.sync_copy(seeds_ref.at[cid, pl.ds(rpt * _NS, tail)],
                                acc.at[pl.ds(rpt * _NS, tail)])
        plsc.subcore_barrier()

        def chunk(j, _):
            off = base + j * _CH
            pltpu.sync_copy(dst_ref.at[pl.ds(off, _CH)], idx_c.at[0])
            pltpu.sync_copy(m_ref.at[pl.ds(off, _CH)], buf)
            pltpu.sync_copy(buf, acc.at[idx_c.at[0]], add=True)
            return 0
        lax.fori_loop(0, n_full, chunk, 0, unroll=False)
        if rem:
            off = base + n_full * _CH
            pltpu.sync_copy(dst_ref.at[pl.ds(off, rem)], idx_r.at[0])
            pltpu.sync_copy(m_ref.at[pl.ds(off, rem)], buf_r)
            pltpu.sync_copy(buf_r, acc.at[idx_r.at[0]], add=True)
        plsc.subcore_barrier()
        pltpu.sync_copy(acc.at[pl.ds(sid * rpt, rpt)],
                        out_ref.at[cid, pl.ds(sid * rpt, rpt)])
        if tail:
            @pl.when(sid == 0)
            def _tail_out():
                pltpu.sync_copy(acc.at[pl.ds(rpt * _NS, tail)],
                                out_ref.at[cid, pl.ds(rpt * _NS, tail)])

    fn = pl.kernel(body,
                   out_type=jax.ShapeDtypeStruct((_NC, Nn, D), jnp.float32),
                   mesh=mesh, scratch_types=scratch,
                   compiler_params=pltpu.CompilerParams(use_tc_tiling_on_sc=False))
    return fn(msgs, dst, seeds)


def kernel(x, edge_index, e, xbatch,
           bn_node_g, bn_node_b, bn_edge_g, bn_edge_b,
           nn1_W0, nn1_b0, nn1_W1, nn1_b1, conv1_root, conv1_bias,
           nn2_W0, nn2_b0, nn2_W1, nn2_b1, conv2_root, conv2_bias,
           mlp_W0, mlp_b0, mlp_W1, mlp_b1, mlp_W2, mlp_b2,
           mlp_W3, mlp_b3, mlp_W4, mlp_b4):
    src = edge_index[0]
    dst = edge_index[1]

    es, eq = _edge_stats(e)
    xn, r1 = _node1(x, bn_node_g, bn_node_b, conv1_root, conv1_bias)

    (xs,) = _sc_gather(xn, src)
    m1 = _msg(e, xs, bn_edge_g, bn_edge_b, es, eq,
              nn1_W0, nn1_b0, nn1_W1, nn1_b1, fan_in=16, fan_out=32)
    seeds1 = jnp.stack([r1, jnp.zeros_like(r1)])
    if _XLA_SCATTER:
        parts1 = jnp.stack([r1 + jnp.zeros_like(r1).at[dst].add(m1),
                            jnp.zeros_like(r1)])
    else:
        parts1 = _sc_scatter(m1, dst, seeds1)
    h1, r2 = _node2(parts1, conv2_root, conv2_bias)

    (h1s,) = _sc_gather(h1, src)
    m2 = _msg(e, h1s, bn_edge_g, bn_edge_b, es, eq,
              nn2_W0, nn2_b0, nn2_W1, nn2_b1, fan_in=32, fan_out=64)
    seeds2 = jnp.stack([r2, jnp.zeros_like(r2)])
    if _XLA_SCATTER:
        parts2 = jnp.stack([r2 + jnp.zeros_like(r2).at[dst].add(m2),
                            jnp.zeros_like(r2)])
    else:
        parts2 = _sc_scatter(m2, dst, seeds2)
    h2 = _hsum(parts2)

    h2s, h2d = _sc_gather(h2, src, dst)
    return _final_mlp(e, h2s, h2d, bn_edge_g, bn_edge_b, es, eq,
                      mlp_W0, mlp_b0, mlp_W1, mlp_b1, mlp_W2, mlp_b2,
                      mlp_W3, mlp_b3, mlp_W4, mlp_b4)


# speedup vs baseline: 1.9398x; 1.0910x over previous
import functools

import jax
import jax.numpy as jnp
from jax import lax
from jax.experimental import pallas as pl
from jax.experimental.pallas import tpu as pltpu
from jax.experimental.pallas import tpu_sc as plsc

_NC, _NS = 2, 16
_NW = _NC * _NS
_CH = 128


_XLA_SCATTER = False
_XLA_GATHER = False


def _dot(a, b):
    return jnp.dot(a.astype(jnp.bfloat16), b.astype(jnp.bfloat16),
                   preferred_element_type=jnp.float32)

def _leaky(v):
    return jnp.where(v >= 0, v, 0.1 * v)


def _estats_body(e_ref, s_ref, q_ref):
    i = pl.program_id(0)

    @pl.when(i == 0)
    def _init():
        s_ref[...] = jnp.zeros_like(s_ref)
        q_ref[...] = jnp.zeros_like(q_ref)

    blk = e_ref[...]
    s_ref[...] += jnp.sum(blk, axis=0, keepdims=True)
    q_ref[...] += jnp.sum(blk * blk, axis=0, keepdims=True)


def _edge_stats(e, tile=8000):
    E, F = e.shape
    return pl.pallas_call(
        _estats_body,
        grid=(E // tile,),
        in_specs=[pl.BlockSpec((tile, F), lambda i: (i, 0))],
        out_specs=[pl.BlockSpec((1, F), lambda i: (0, 0))] * 2,
        out_shape=[jax.ShapeDtypeStruct((1, F), jnp.float32)] * 2,
    )(e)


def _node1_body(x_ref, g_ref, b_ref, root_ref, bias_ref, xn_ref, r1_ref):
    x = x_ref[...]
    m = jnp.mean(x, axis=0, keepdims=True)
    var = jnp.mean(x * x, axis=0, keepdims=True) - m * m
    xn = (x - m) * (g_ref[...] * lax.rsqrt(var + 1e-5)) + b_ref[...]
    xn_ref[...] = xn
    r1_ref[...] = (
        _dot(xn, root_ref[...])
        + bias_ref[...]
    )


def _node1(x, g, b, root, bias):
    N, F = x.shape
    Fo = root.shape[1]
    return pl.pallas_call(
        _node1_body,
        out_shape=[
            jax.ShapeDtypeStruct((N, F), jnp.float32),
            jax.ShapeDtypeStruct((N, Fo), jnp.float32),
        ],
    )(x, g.reshape(1, -1), b.reshape(1, -1), root, bias.reshape(1, -1))


def _node2_body(parts_ref, root_ref, bias_ref, h_ref, r_ref):
    h = parts_ref[0] + parts_ref[1]
    h_ref[...] = h
    r_ref[...] = (
        _dot(h, root_ref[...])
        + bias_ref[...]
    )


def _node2(parts, root, bias):
    _, N, F = parts.shape
    Fo = root.shape[1]
    return pl.pallas_call(
        _node2_body,
        out_shape=[
            jax.ShapeDtypeStruct((N, F), jnp.float32),
            jax.ShapeDtypeStruct((N, Fo), jnp.float32),
        ],
    )(parts, root, bias.reshape(1, -1))


def _hsum_body(parts_ref, h_ref):
    h_ref[...] = parts_ref[0] + parts_ref[1]


def _hsum(parts):
    _, N, F = parts.shape
    return pl.pallas_call(
        _hsum_body,
        out_shape=jax.ShapeDtypeStruct((N, F), jnp.float32),
    )(parts)


def _make_msg_body(E, fan_in, fan_out):
    def body(e_ref, xs_ref, sel_ref, eg_ref, eb_ref, s_ref, q_ref,
             w0_ref, b0_ref, w1_ref, b1_ref, out_ref):
        mean = s_ref[...] / E
        var = q_ref[...] / E - mean * mean
        en = (e_ref[...] - mean) * (eg_ref[...] * lax.rsqrt(var + 1e-5)) + eb_ref[...]
        u = _leaky(_dot(en, w0_ref[...]) + b0_ref[...])
        w = _leaky(_dot(u, w1_ref[...]) + b1_ref[...])
        w = w.astype(jnp.bfloat16).astype(jnp.float32)
        xs = xs_ref[...].astype(jnp.bfloat16).astype(jnp.float32)
        xs_rep = _dot(xs, sel_ref[...])
        acc = xs_rep * w
        while acc.shape[1] > fan_out:
            half = acc.shape[1] // 2
            acc = acc[:, :half] + acc[:, half:]
        out_ref[...] = acc
    return body


def _msg(e, xs, eg, eb, es, eq, w0, b0, w1, b1, fan_in, fan_out, tile=1000):
    E, F = e.shape
    fhid = w0.shape[1]
    body = _make_msg_body(E, fan_in, fan_out)
    wide = w1.shape[1]
    return pl.pallas_call(
        body,
        grid=(E // tile,),
        in_specs=[
            pl.BlockSpec((tile, F), lambda i: (i, 0)),
            pl.BlockSpec((tile, fan_in), lambda i: (i, 0)),
            pl.BlockSpec((fan_in, wide), lambda i: (0, 0)),
            pl.BlockSpec((1, F), lambda i: (0, 0)),
            pl.BlockSpec((1, F), lambda i: (0, 0)),
            pl.BlockSpec((1, F), lambda i: (0, 0)),
            pl.BlockSpec((1, F), lambda i: (0, 0)),
            pl.BlockSpec((F, fhid), lambda i: (0, 0)),
            pl.BlockSpec((1, fhid), lambda i: (0, 0)),
            pl.BlockSpec((fhid, wide), lambda i: (0, 0)),
            pl.BlockSpec((1, wide), lambda i: (0, 0)),
        ],
        out_specs=pl.BlockSpec((tile, fan_out), lambda i: (i, 0)),
        out_shape=jax.ShapeDtypeStruct((E, fan_out), jnp.float32),
    )(e, xs, jnp.repeat(jnp.eye(fan_in, dtype=jnp.float32), fan_out, axis=1),
      eg.reshape(1, -1), eb.reshape(1, -1), es, eq,
      w0, b0.reshape(1, -1), w1, b1.reshape(1, -1))


def _make_final_body(E):
    def body(e_ref, hs_ref, hd_ref, eg_ref, eb_ref, s_ref, q_ref,
             w0a_ref, w0b_ref, w0c_ref, b0_ref, w1_ref, b1_ref,
             w2_ref, b2_ref, w3_ref, b3_ref, w4_ref, b4_ref, out_ref):
        mean = s_ref[...] / E
        var = q_ref[...] / E - mean * mean
        en = (e_ref[...] - mean) * (eg_ref[...] * lax.rsqrt(var + 1e-5)) + eb_ref[...]
        t = _leaky(_dot(hs_ref[...], w0a_ref[...])
                   + _dot(hd_ref[...], w0b_ref[...])
                   + _dot(en, w0c_ref[...])
                   + b0_ref[...])
        t = _leaky(_dot(t, w1_ref[...]) + b1_ref[...])
        t = _leaky(_dot(t, w2_ref[...]) + b2_ref[...])
        t = _leaky(_dot(t, w3_ref[...]) + b3_ref[...])
        out_ref[...] = _dot(t, w4_ref[...]) + b4_ref[...]
    return body


def _final_mlp(e, hs, hd, eg, eb, es, eq, w0, b0, w1, b1, w2, b2, w3, b3,
               w4, b4, tile=1000):
    E, F = e.shape
    H = hs.shape[1]
    w0a, w0b, w0c = w0[:H], w0[H:2 * H], w0[2 * H:]
    full = lambda a: pl.BlockSpec(a.shape, lambda i: tuple(0 for _ in a.shape))
    b0r, b1r, b2r, b3r, b4r = (v.reshape(1, -1) for v in (b0, b1, b2, b3, b4))
    args = (e, hs, hd, eg.reshape(1, -1), eb.reshape(1, -1), es, eq,
            w0a, w0b, w0c, b0r, w1, b1r, w2, b2r, w3, b3r, w4, b4r)
    in_specs = [
        pl.BlockSpec((tile, F), lambda i: (i, 0)),
        pl.BlockSpec((tile, H), lambda i: (i, 0)),
        pl.BlockSpec((tile, H), lambda i: (i, 0)),
    ] + [full(a) for a in args[3:]]
    return pl.pallas_call(
        _make_final_body(E),
        grid=(E // tile,),
        in_specs=in_specs,
        out_specs=pl.BlockSpec((tile, 2), lambda i: (i, 0)),
        out_shape=jax.ShapeDtypeStruct((E, 2), jnp.float32),
    )(*args)


def _sc_gather(table, *idxs):
    if _XLA_GATHER:
        return tuple(jnp.take(table, i, axis=0) for i in idxs)
    Nn, D = table.shape
    E = idxs[0].shape[0]
    per_w = E // _NW
    n_full, rem = divmod(per_w, _CH)
    pairs, odd = divmod(n_full, 2)
    n_idx = len(idxs)
    mesh = plsc.VectorSubcoreMesh(core_axis_name="c", subcore_axis_name="s",
                                  num_cores=_NC, num_subcores=_NS)
    scratch = [
        pltpu.VMEM((per_w,), jnp.int32),
        pltpu.VMEM((_CH, D), jnp.float32),
        pltpu.VMEM((_CH, D), jnp.float32),
        pltpu.VMEM((max(rem, 1), D), jnp.float32),
        pltpu.SemaphoreType.DMA,
        pltpu.SemaphoreType.DMA,
        pltpu.SemaphoreType.DMA,
        pltpu.SemaphoreType.DMA,
    ]

    def body(table_ref, *rest):
        idx_refs = rest[:n_idx]
        out_refs = rest[n_idx:2 * n_idx]
        idx_all, buf0, buf1, buf_r, g0, g1, s0, s1 = rest[2 * n_idx:]
        bufs, gsem, ssem = (buf0, buf1), (g0, g1), (s0, s1)
        wid = lax.axis_index("s") * _NC + lax.axis_index("c")
        base = wid * per_w

        for k in range(n_idx):
            out_ref = out_refs[k]

            def drain_store(b, k=k):
                pltpu.make_async_copy(
                    bufs[b], out_refs[k].at[pl.ds(base, _CH)], ssem[b]).wait()

            pltpu.sync_copy(idx_refs[k].at[pl.ds(base, per_w)], idx_all)

            def pair(j2, _, k=k):
                for b in range(2):
                    j = j2 * 2 + b

                    @pl.when(j2 > 0)
                    def _(b=b):
                        drain_store(b)
                    off = base + j * _CH
                    pltpu.async_copy(
                        table_ref.at[idx_all.at[pl.ds(j * _CH, _CH)]],
                        bufs[b], gsem[b]).wait()
                    pltpu.async_copy(bufs[b], out_refs[k].at[pl.ds(off, _CH)],
                                     ssem[b])
                return 0
            lax.fori_loop(0, pairs, pair, 0, unroll=False)
            if odd:
                j = pairs * 2
                if pairs > 0:
                    drain_store(0)
                pltpu.async_copy(
                    table_ref.at[idx_all.at[pl.ds(j * _CH, _CH)]],
                    bufs[0], gsem[0]).wait()
                pltpu.async_copy(bufs[0], out_ref.at[pl.ds(base + j * _CH, _CH)],
                                 ssem[0])
            if rem:
                off = base + n_full * _CH
                pltpu.async_copy(
                    table_ref.at[idx_all.at[pl.ds(n_full * _CH, rem)]],
                    buf_r, gsem[1]).wait()
                pltpu.sync_copy(buf_r, out_ref.at[pl.ds(off, rem)])
            if n_full > 0:
                drain_store(odd)
            if n_full > 1 or (odd and n_full > 0):
                drain_store(1 - odd)

    out_type = tuple(jax.ShapeDtypeStruct((E, D), jnp.float32)
                     for _ in range(n_idx))
    fn = pl.kernel(body, out_type=out_type, mesh=mesh, scratch_types=scratch,
                   compiler_params=pltpu.CompilerParams(use_tc_tiling_on_sc=False))
    return fn(table, *idxs)


def _sc_scatter(msgs, dst, seeds):
    E, D = msgs.shape
    Nn = seeds.shape[1]
    per_w = E // _NW
    n_full, rem = divmod(per_w, _CH)
    rpt = (Nn // _NS) // 8 * 8
    tail = Nn - rpt * _NS
    mesh = plsc.VectorSubcoreMesh(core_axis_name="c", subcore_axis_name="s",
                                  num_cores=_NC, num_subcores=_NS)
    pairs, odd = divmod(n_full, 2)
    scratch = [
        pltpu.VMEM((2, _CH), jnp.int32),
        pltpu.VMEM((_CH, D), jnp.float32),
        pltpu.VMEM((_CH, D), jnp.float32),
        pltpu.VMEM((1, max(rem, 1)), jnp.int32),
        pltpu.VMEM((max(rem, 1), D), jnp.float32),
        pltpu.VMEM_SHARED((Nn, D), jnp.float32),
        pltpu.SemaphoreType.DMA,
        pltpu.SemaphoreType.DMA,
    ]

    def body(m_ref, dst_ref, seeds_ref, out_ref, idx2, buf0, buf1,
             idx_r, buf_r, acc, t0, t1):
        bufs, tsem = (buf0, buf1), (t0, t1)
        cid = lax.axis_index("c")
        sid = lax.axis_index("s")
        wid = sid * _NC + cid
        base = wid * per_w
        pltpu.sync_copy(seeds_ref.at[cid, pl.ds(sid * rpt, rpt)],
                        acc.at[pl.ds(sid * rpt, rpt)])
        if tail:
            @pl.when(sid == 0)
            def _tail_init():
                pltpu.sync_copy(seeds_ref.at[cid, pl.ds(rpt * _NS, tail)],
                                acc.at[pl.ds(rpt * _NS, tail)])
        plsc.subcore_barrier()

        def stage(j, b):
            off = base + j * _CH
            pltpu.async_copy(dst_ref.at[pl.ds(off, _CH)], idx2.at[b], tsem[b])
            pltpu.async_copy(m_ref.at[pl.ds(off, _CH)], bufs[b], tsem[b])

        def wait_stage(b):
            pltpu.make_async_copy(dst_ref.at[pl.ds(base, _CH)], idx2.at[b],
                                  tsem[b]).wait()
            pltpu.make_async_copy(m_ref.at[pl.ds(base, _CH)], bufs[b],
                                  tsem[b]).wait()

        if n_full > 0:
            stage(0, 0)

        def pair(j2, _):
            for b in range(2):
                j = j2 * 2 + b
                nxt = j + 1
                if odd:
                    stage(nxt, 1 - b)
                else:
                    @pl.when(nxt < n_full)
                    def _(nxt=nxt, b=b):
                        stage(nxt, 1 - b)
                wait_stage(b)
                pltpu.sync_copy(bufs[b], acc.at[idx2.at[b]], add=True)
            return 0
        lax.fori_loop(0, pairs, pair, 0, unroll=False)
        if odd:
            wait_stage(0)
            pltpu.sync_copy(bufs[0], acc.at[idx2.at[0]], add=True)
        if rem:
            off = base + n_full * _CH
            pltpu.sync_copy(dst_ref.at[pl.ds(off, rem)], idx_r.at[0])
            pltpu.sync_copy(m_ref.at[pl.ds(off, rem)], buf_r)
            pltpu.sync_copy(buf_r, acc.at[idx_r.at[0]], add=True)
        plsc.subcore_barrier()
        pltpu.sync_copy(acc.at[pl.ds(sid * rpt, rpt)],
                        out_ref.at[cid, pl.ds(sid * rpt, rpt)])
        if tail:
            @pl.when(sid == 0)
            def _tail_out():
                pltpu.sync_copy(acc.at[pl.ds(rpt * _NS, tail)],
                                out_ref.at[cid, pl.ds(rpt * _NS, tail)])

    fn = pl.kernel(body,
                   out_type=jax.ShapeDtypeStruct((_NC, Nn, D), jnp.float32),
                   mesh=mesh, scratch_types=scratch,
                   compiler_params=pltpu.CompilerParams(use_tc_tiling_on_sc=False))
    return fn(msgs, dst, seeds)


def kernel(x, edge_index, e, xbatch,
           bn_node_g, bn_node_b, bn_edge_g, bn_edge_b,
           nn1_W0, nn1_b0, nn1_W1, nn1_b1, conv1_root, conv1_bias,
           nn2_W0, nn2_b0, nn2_W1, nn2_b1, conv2_root, conv2_bias,
           mlp_W0, mlp_b0, mlp_W1, mlp_b1, mlp_W2, mlp_b2,
           mlp_W3, mlp_b3, mlp_W4, mlp_b4):
    src = edge_index[0]
    dst = edge_index[1]

    es, eq = _edge_stats(e)
    xn, r1 = _node1(x, bn_node_g, bn_node_b, conv1_root, conv1_bias)

    (xs,) = _sc_gather(xn, src)
    m1 = _msg(e, xs, bn_edge_g, bn_edge_b, es, eq,
              nn1_W0, nn1_b0, nn1_W1, nn1_b1, fan_in=16, fan_out=32)
    seeds1 = jnp.stack([r1, jnp.zeros_like(r1)])
    if _XLA_SCATTER:
        parts1 = jnp.stack([r1 + jnp.zeros_like(r1).at[dst].add(m1),
                            jnp.zeros_like(r1)])
    else:
        parts1 = _sc_scatter(m1, dst, seeds1)
    h1, r2 = _node2(parts1, conv2_root, conv2_bias)

    (h1s,) = _sc_gather(h1, src)
    m2 = _msg(e, h1s, bn_edge_g, bn_edge_b, es, eq,
              nn2_W0, nn2_b0, nn2_W1, nn2_b1, fan_in=32, fan_out=64)
    seeds2 = jnp.stack([r2, jnp.zeros_like(r2)])
    if _XLA_SCATTER:
        parts2 = jnp.stack([r2 + jnp.zeros_like(r2).at[dst].add(m2),
                            jnp.zeros_like(r2)])
    else:
        parts2 = _sc_scatter(m2, dst, seeds2)
    h2 = _hsum(parts2)

    h2s, h2d = _sc_gather(h2, src, dst)
    return _final_mlp(e, h2s, h2d, bn_edge_g, bn_edge_b, es, eq,
                      mlp_W0, mlp_b0, mlp_W1, mlp_b1, mlp_W2, mlp_b2,
                      mlp_W3, mlp_b3, mlp_W4, mlp_b4)


# speedup vs baseline: 2.2323x; 1.1508x over previous
import functools

import jax
import jax.numpy as jnp
from jax import lax
from jax.experimental import pallas as pl
from jax.experimental.pallas import tpu as pltpu
from jax.experimental.pallas import tpu_sc as plsc

_NC, _NS = 2, 16
_NW = _NC * _NS
_CH = 128


def _dot(a, b):
    return jnp.dot(a.astype(jnp.bfloat16), b.astype(jnp.bfloat16),
                   preferred_element_type=jnp.float32)

def _leaky(v):
    return jnp.where(v >= 0, v, 0.1 * v)


def _estats_body(e_ref, s_ref, q_ref):
    i = pl.program_id(0)

    @pl.when(i == 0)
    def _init():
        s_ref[...] = jnp.zeros_like(s_ref)
        q_ref[...] = jnp.zeros_like(q_ref)

    blk = e_ref[...]
    s_ref[...] += jnp.sum(blk, axis=0, keepdims=True)
    q_ref[...] += jnp.sum(blk * blk, axis=0, keepdims=True)


def _edge_stats(e, tile=16000):
    E, F = e.shape
    return pl.pallas_call(
        _estats_body,
        grid=(E // tile,),
        in_specs=[pl.BlockSpec((tile, F), lambda i: (i, 0))],
        out_specs=[pl.BlockSpec((1, F), lambda i: (0, 0))] * 2,
        out_shape=[jax.ShapeDtypeStruct((1, F), jnp.float32)] * 2,
    )(e)


def _node1_body(x_ref, g_ref, b_ref, root_ref, bias_ref, xn_ref, r1_ref):
    x = x_ref[...]
    m = jnp.mean(x, axis=0, keepdims=True)
    var = jnp.mean(x * x, axis=0, keepdims=True) - m * m
    xn = (x - m) * (g_ref[...] * lax.rsqrt(var + 1e-5)) + b_ref[...]
    xn_ref[...] = xn
    r1_ref[...] = (
        _dot(xn, root_ref[...])
        + bias_ref[...]
    )


def _node1(x, g, b, root, bias):
    N, F = x.shape
    Fo = root.shape[1]
    return pl.pallas_call(
        _node1_body,
        out_shape=[
            jax.ShapeDtypeStruct((N, F), jnp.float32),
            jax.ShapeDtypeStruct((N, Fo), jnp.float32),
        ],
    )(x, g.reshape(1, -1), b.reshape(1, -1), root, bias.reshape(1, -1))


def _node2_body(parts_ref, root_ref, bias_ref, h_ref, r_ref):
    h = parts_ref[0] + parts_ref[1]
    h_ref[...] = h
    r_ref[...] = (
        _dot(h, root_ref[...])
        + bias_ref[...]
    )


def _node2(parts, root, bias):
    _, N, F = parts.shape
    Fo = root.shape[1]
    return pl.pallas_call(
        _node2_body,
        out_shape=[
            jax.ShapeDtypeStruct((N, F), jnp.float32),
            jax.ShapeDtypeStruct((N, Fo), jnp.float32),
        ],
    )(parts, root, bias.reshape(1, -1))


def _hsum_body(parts_ref, h_ref):
    h_ref[...] = parts_ref[0] + parts_ref[1]


def _hsum(parts):
    _, N, F = parts.shape
    return pl.pallas_call(
        _hsum_body,
        out_shape=jax.ShapeDtypeStruct((N, F), jnp.float32),
    )(parts)


def _make_msg_body(E, fan_in, fan_out):
    def body(e_ref, xs_ref, sel_ref, eg_ref, eb_ref, s_ref, q_ref,
             w0_ref, b0_ref, w1_ref, b1_ref, out_ref):
        mean = s_ref[...] / E
        var = q_ref[...] / E - mean * mean
        en = (e_ref[...] - mean) * (eg_ref[...] * lax.rsqrt(var + 1e-5)) + eb_ref[...]
        u = _leaky(_dot(en, w0_ref[...]) + b0_ref[...])
        w = _leaky(_dot(u, w1_ref[...]) + b1_ref[...])
        w = w.astype(jnp.bfloat16).astype(jnp.float32)
        xs = xs_ref[...].astype(jnp.bfloat16).astype(jnp.float32)
        xs_rep = _dot(xs, sel_ref[...])
        acc = xs_rep * w
        while acc.shape[1] > fan_out:
            half = acc.shape[1] // 2
            acc = acc[:, :half] + acc[:, half:]
        out_ref[...] = acc
    return body


def _msg(e, xs, eg, eb, es, eq, w0, b0, w1, b1, fan_in, fan_out, tile=2000):
    E, F = e.shape
    fhid = w0.shape[1]
    body = _make_msg_body(E, fan_in, fan_out)
    wide = w1.shape[1]
    return pl.pallas_call(
        body,
        grid=(E // tile,),
        in_specs=[
            pl.BlockSpec((tile, F), lambda i: (i, 0)),
            pl.BlockSpec((tile, fan_in), lambda i: (i, 0)),
            pl.BlockSpec((fan_in, wide), lambda i: (0, 0)),
            pl.BlockSpec((1, F), lambda i: (0, 0)),
            pl.BlockSpec((1, F), lambda i: (0, 0)),
            pl.BlockSpec((1, F), lambda i: (0, 0)),
            pl.BlockSpec((1, F), lambda i: (0, 0)),
            pl.BlockSpec((F, fhid), lambda i: (0, 0)),
            pl.BlockSpec((1, fhid), lambda i: (0, 0)),
            pl.BlockSpec((fhid, wide), lambda i: (0, 0)),
            pl.BlockSpec((1, wide), lambda i: (0, 0)),
        ],
        out_specs=pl.BlockSpec((tile, fan_out), lambda i: (i, 0)),
        out_shape=jax.ShapeDtypeStruct((E, fan_out), jnp.float32),
    )(e, xs, jnp.repeat(jnp.eye(fan_in, dtype=jnp.float32), fan_out, axis=1),
      eg.reshape(1, -1), eb.reshape(1, -1), es, eq,
      w0, b0.reshape(1, -1), w1, b1.reshape(1, -1))


def _make_final_body(E):
    def body(e_ref, hs_ref, hd_ref, eg_ref, eb_ref, s_ref, q_ref,
             w0a_ref, w0b_ref, w0c_ref, b0_ref, w1_ref, b1_ref,
             w2_ref, b2_ref, w3_ref, b3_ref, w4_ref, b4_ref, out_ref):
        mean = s_ref[...] / E
        var = q_ref[...] / E - mean * mean
        en = (e_ref[...] - mean) * (eg_ref[...] * lax.rsqrt(var + 1e-5)) + eb_ref[...]
        t = _leaky(_dot(hs_ref[...], w0a_ref[...])
                   + _dot(hd_ref[...], w0b_ref[...])
                   + _dot(en, w0c_ref[...])
                   + b0_ref[...])
        t = _leaky(_dot(t, w1_ref[...]) + b1_ref[...])
        t = _leaky(_dot(t, w2_ref[...]) + b2_ref[...])
        t = _leaky(_dot(t, w3_ref[...]) + b3_ref[...])
        out_ref[...] = _dot(t, w4_ref[...]) + b4_ref[...]
    return body


def _final_mlp(e, hs, hd, eg, eb, es, eq, w0, b0, w1, b1, w2, b2, w3, b3,
               w4, b4, tile=4000):
    E, F = e.shape
    H = hs.shape[1]
    w0a, w0b, w0c = w0[:H], w0[H:2 * H], w0[2 * H:]
    full = lambda a: pl.BlockSpec(a.shape, lambda i: tuple(0 for _ in a.shape))
    b0r, b1r, b2r, b3r, b4r = (v.reshape(1, -1) for v in (b0, b1, b2, b3, b4))
    args = (e, hs, hd, eg.reshape(1, -1), eb.reshape(1, -1), es, eq,
            w0a, w0b, w0c, b0r, w1, b1r, w2, b2r, w3, b3r, w4, b4r)
    in_specs = [
        pl.BlockSpec((tile, F), lambda i: (i, 0)),
        pl.BlockSpec((tile, H), lambda i: (i, 0)),
        pl.BlockSpec((tile, H), lambda i: (i, 0)),
    ] + [full(a) for a in args[3:]]
    return pl.pallas_call(
        _make_final_body(E),
        grid=(E // tile,),
        in_specs=in_specs,
        out_specs=pl.BlockSpec((tile, 2), lambda i: (i, 0)),
        out_shape=jax.ShapeDtypeStruct((E, 2), jnp.float32),
    )(*args)


def _sc_gather(table, *idxs):
    Nn, D = table.shape
    E = idxs[0].shape[0]
    per_w = E // _NW
    n_full, rem = divmod(per_w, _CH)
    pairs, odd = divmod(n_full, 2)
    n_idx = len(idxs)
    mesh = plsc.VectorSubcoreMesh(core_axis_name="c", subcore_axis_name="s",
                                  num_cores=_NC, num_subcores=_NS)
    scratch = [
        pltpu.VMEM((per_w,), jnp.int32),
        pltpu.VMEM((_CH, D), jnp.float32),
        pltpu.VMEM((_CH, D), jnp.float32),
        pltpu.VMEM((max(rem, 1), D), jnp.float32),
        pltpu.SemaphoreType.DMA,
        pltpu.SemaphoreType.DMA,
        pltpu.SemaphoreType.DMA,
        pltpu.SemaphoreType.DMA,
    ]

    def body(table_ref, *rest):
        idx_refs = rest[:n_idx]
        out_refs = rest[n_idx:2 * n_idx]
        idx_all, buf0, buf1, buf_r, g0, g1, s0, s1 = rest[2 * n_idx:]
        bufs, gsem, ssem = (buf0, buf1), (g0, g1), (s0, s1)
        wid = lax.axis_index("s") * _NC + lax.axis_index("c")
        base = wid * per_w

        for k in range(n_idx):
            out_ref = out_refs[k]

            def drain_store(b, k=k):
                pltpu.make_async_copy(
                    bufs[b], out_refs[k].at[pl.ds(base, _CH)], ssem[b]).wait()

            pltpu.sync_copy(idx_refs[k].at[pl.ds(base, per_w)], idx_all)

            def pair(j2, _, k=k):
                for b in range(2):
                    j = j2 * 2 + b

                    @pl.when(j2 > 0)
                    def _(b=b):
                        drain_store(b)
                    off = base + j * _CH
                    pltpu.async_copy(
                        table_ref.at[idx_all.at[pl.ds(j * _CH, _CH)]],
                        bufs[b], gsem[b]).wait()
                    pltpu.async_copy(bufs[b], out_refs[k].at[pl.ds(off, _CH)],
                                     ssem[b])
                return 0
            lax.fori_loop(0, pairs, pair, 0, unroll=False)
            if odd:
                j = pairs * 2
                if pairs > 0:
                    drain_store(0)
                pltpu.async_copy(
                    table_ref.at[idx_all.at[pl.ds(j * _CH, _CH)]],
                    bufs[0], gsem[0]).wait()
                pltpu.async_copy(bufs[0], out_ref.at[pl.ds(base + j * _CH, _CH)],
                                 ssem[0])
            if rem:
                off = base + n_full * _CH
                pltpu.async_copy(
                    table_ref.at[idx_all.at[pl.ds(n_full * _CH, rem)]],
                    buf_r, gsem[1]).wait()
                pltpu.sync_copy(buf_r, out_ref.at[pl.ds(off, rem)])
            if n_full > 0:
                drain_store(odd)
            if n_full > 1 or (odd and n_full > 0):
                drain_store(1 - odd)

    out_type = tuple(jax.ShapeDtypeStruct((E, D), jnp.float32)
                     for _ in range(n_idx))
    fn = pl.kernel(body, out_type=out_type, mesh=mesh, scratch_types=scratch,
                   compiler_params=pltpu.CompilerParams(use_tc_tiling_on_sc=False))
    return fn(table, *idxs)


def _sc_scatter(msgs, dst, seeds):
    E, D = msgs.shape
    Nn = seeds.shape[1]
    per_w = E // _NW
    n_full, rem = divmod(per_w, _CH)
    rpt = (Nn // _NS) // 8 * 8
    tail = Nn - rpt * _NS
    mesh = plsc.VectorSubcoreMesh(core_axis_name="c", subcore_axis_name="s",
                                  num_cores=_NC, num_subcores=_NS)
    pairs, odd = divmod(n_full, 2)
    scratch = [
        pltpu.VMEM((2, _CH), jnp.int32),
        pltpu.VMEM((_CH, D), jnp.float32),
        pltpu.VMEM((_CH, D), jnp.float32),
        pltpu.VMEM((1, max(rem, 1)), jnp.int32),
        pltpu.VMEM((max(rem, 1), D), jnp.float32),
        pltpu.VMEM_SHARED((Nn, D), jnp.float32),
        pltpu.SemaphoreType.DMA,
        pltpu.SemaphoreType.DMA,
    ]

    def body(m_ref, dst_ref, seeds_ref, out_ref, idx2, buf0, buf1,
             idx_r, buf_r, acc, t0, t1):
        bufs, tsem = (buf0, buf1), (t0, t1)
        cid = lax.axis_index("c")
        sid = lax.axis_index("s")
        wid = sid * _NC + cid
        base = wid * per_w
        pltpu.sync_copy(seeds_ref.at[cid, pl.ds(sid * rpt, rpt)],
                        acc.at[pl.ds(sid * rpt, rpt)])
        if tail:
            @pl.when(sid == 0)
            def _tail_init():
                pltpu.sync_copy(seeds_ref.at[cid, pl.ds(rpt * _NS, tail)],
                                acc.at[pl.ds(rpt * _NS, tail)])
        plsc.subcore_barrier()

        def stage(j, b):
            off = base + j * _CH
            pltpu.async_copy(dst_ref.at[pl.ds(off, _CH)], idx2.at[b], tsem[b])
            pltpu.async_copy(m_ref.at[pl.ds(off, _CH)], bufs[b], tsem[b])

        def wait_stage(b):
            pltpu.make_async_copy(dst_ref.at[pl.ds(base, _CH)], idx2.at[b],
                                  tsem[b]).wait()
            pltpu.make_async_copy(m_ref.at[pl.ds(base, _CH)], bufs[b],
                                  tsem[b]).wait()

        if n_full > 0:
            stage(0, 0)

        def pair(j2, _):
            for b in range(2):
                j = j2 * 2 + b
                nxt = j + 1
                if odd:
                    stage(nxt, 1 - b)
                else:
                    @pl.when(nxt < n_full)
                    def _(nxt=nxt, b=b):
                        stage(nxt, 1 - b)
                wait_stage(b)
                pltpu.sync_copy(bufs[b], acc.at[idx2.at[b]], add=True)
            return 0
        lax.fori_loop(0, pairs, pair, 0, unroll=False)
        if odd:
            wait_stage(0)
            pltpu.sync_copy(bufs[0], acc.at[idx2.at[0]], add=True)
        if rem:
            off = base + n_full * _CH
            pltpu.sync_copy(dst_ref.at[pl.ds(off, rem)], idx_r.at[0])
            pltpu.sync_copy(m_ref.at[pl.ds(off, rem)], buf_r)
            pltpu.sync_copy(buf_r, acc.at[idx_r.at[0]], add=True)
        plsc.subcore_barrier()
        pltpu.sync_copy(acc.at[pl.ds(sid * rpt, rpt)],
                        out_ref.at[cid, pl.ds(sid * rpt, rpt)])
        if tail:
            @pl.when(sid == 0)
            def _tail_out():
                pltpu.sync_copy(acc.at[pl.ds(rpt * _NS, tail)],
                                out_ref.at[cid, pl.ds(rpt * _NS, tail)])

    fn = pl.kernel(body,
                   out_type=jax.ShapeDtypeStruct((_NC, Nn, D), jnp.float32),
                   mesh=mesh, scratch_types=scratch,
                   compiler_params=pltpu.CompilerParams(use_tc_tiling_on_sc=False))
    return fn(msgs, dst, seeds)


def kernel(x, edge_index, e, xbatch,
           bn_node_g, bn_node_b, bn_edge_g, bn_edge_b,
           nn1_W0, nn1_b0, nn1_W1, nn1_b1, conv1_root, conv1_bias,
           nn2_W0, nn2_b0, nn2_W1, nn2_b1, conv2_root, conv2_bias,
           mlp_W0, mlp_b0, mlp_W1, mlp_b1, mlp_W2, mlp_b2,
           mlp_W3, mlp_b3, mlp_W4, mlp_b4):
    src = edge_index[0]
    dst = edge_index[1]

    es, eq = _edge_stats(e)
    xn, r1 = _node1(x, bn_node_g, bn_node_b, conv1_root, conv1_bias)

    (xs,) = _sc_gather(xn, src)
    m1 = _msg(e, xs, bn_edge_g, bn_edge_b, es, eq,
              nn1_W0, nn1_b0, nn1_W1, nn1_b1, fan_in=16, fan_out=32,
              tile=4000)
    seeds1 = jnp.stack([r1, jnp.zeros_like(r1)])
    parts1 = _sc_scatter(m1, dst, seeds1)
    h1, r2 = _node2(parts1, conv2_root, conv2_bias)

    (h1s,) = _sc_gather(h1, src)
    m2 = _msg(e, h1s, bn_edge_g, bn_edge_b, es, eq,
              nn2_W0, nn2_b0, nn2_W1, nn2_b1, fan_in=32, fan_out=64,
              tile=3200)
    seeds2 = jnp.stack([r2, jnp.zeros_like(r2)])
    parts2 = _sc_scatter(m2, dst, seeds2)
    h2 = _hsum(parts2)

    h2s, h2d = _sc_gather(h2, src, dst)
    return _final_mlp(e, h2s, h2d, bn_edge_g, bn_edge_b, es, eq,
                      mlp_W0, mlp_b0, mlp_W1, mlp_b1, mlp_W2, mlp_b2,
                      mlp_W3, mlp_b3, mlp_W4, mlp_b4)


# speedup vs baseline: 2.2504x; 1.0081x over previous
import functools

import jax
import jax.numpy as jnp
from jax import lax
from jax.experimental import pallas as pl
from jax.experimental.pallas import tpu as pltpu
from jax.experimental.pallas import tpu_sc as plsc

_NC, _NS = 2, 16
_NW = _NC * _NS
_CH = 128


def _dot(a, b):
    return jnp.dot(a.astype(jnp.bfloat16), b.astype(jnp.bfloat16),
                   preferred_element_type=jnp.float32)

def _leaky(v):
    return jnp.where(v >= 0, v, 0.1 * v)


def _estats_body(e_ref, s_ref, q_ref):
    i = pl.program_id(0)

    @pl.when(i == 0)
    def _init():
        s_ref[...] = jnp.zeros_like(s_ref)
        q_ref[...] = jnp.zeros_like(q_ref)

    blk = e_ref[...]
    s_ref[...] += jnp.sum(blk, axis=0, keepdims=True)
    q_ref[...] += jnp.sum(blk * blk, axis=0, keepdims=True)


def _edge_stats(e, tile=16000):
    E, F = e.shape
    return pl.pallas_call(
        _estats_body,
        grid=(E // tile,),
        in_specs=[pl.BlockSpec((tile, F), lambda i: (i, 0))],
        out_specs=[pl.BlockSpec((1, F), lambda i: (0, 0))] * 2,
        out_shape=[jax.ShapeDtypeStruct((1, F), jnp.float32)] * 2,
    )(e)


def _node1_body(x_ref, g_ref, b_ref, root_ref, bias_ref, xn_ref, r1_ref):
    x = x_ref[...]
    m = jnp.mean(x, axis=0, keepdims=True)
    var = jnp.mean(x * x, axis=0, keepdims=True) - m * m
    xn = (x - m) * (g_ref[...] * lax.rsqrt(var + 1e-5)) + b_ref[...]
    xn_ref[...] = xn
    r1_ref[...] = (
        _dot(xn, root_ref[...])
        + bias_ref[...]
    )


def _node1(x, g, b, root, bias):
    N, F = x.shape
    Fo = root.shape[1]
    return pl.pallas_call(
        _node1_body,
        out_shape=[
            jax.ShapeDtypeStruct((N, F), jnp.float32),
            jax.ShapeDtypeStruct((N, Fo), jnp.float32),
        ],
    )(x, g.reshape(1, -1), b.reshape(1, -1), root, bias.reshape(1, -1))


def _node2_body(parts_ref, root_ref, bias_ref, h_ref, r_ref):
    h = parts_ref[0] + parts_ref[1]
    h_ref[...] = h
    r_ref[...] = (
        _dot(h, root_ref[...])
        + bias_ref[...]
    )


def _node2(parts, root, bias):
    _, N, F = parts.shape
    Fo = root.shape[1]
    return pl.pallas_call(
        _node2_body,
        out_shape=[
            jax.ShapeDtypeStruct((N, F), jnp.float32),
            jax.ShapeDtypeStruct((N, Fo), jnp.float32),
        ],
    )(parts, root, bias.reshape(1, -1))


def _hsum_body(parts_ref, h_ref):
    h_ref[...] = parts_ref[0] + parts_ref[1]


def _hsum(parts):
    _, N, F = parts.shape
    return pl.pallas_call(
        _hsum_body,
        out_shape=jax.ShapeDtypeStruct((N, F), jnp.float32),
    )(parts)


def _make_msg_body(E, fan_in, fan_out):
    def body(e_ref, xs_ref, sel_ref, eg_ref, eb_ref, s_ref, q_ref,
             w0_ref, b0_ref, w1_ref, b1_ref, out_ref):
        mean = s_ref[...] / E
        var = q_ref[...] / E - mean * mean
        en = (e_ref[...] - mean) * (eg_ref[...] * lax.rsqrt(var + 1e-5)) + eb_ref[...]
        u = _leaky(_dot(en, w0_ref[...]) + b0_ref[...])
        w = _leaky(_dot(u, w1_ref[...]) + b1_ref[...])
        w = w.astype(jnp.bfloat16).astype(jnp.float32)
        xs = xs_ref[...].astype(jnp.bfloat16).astype(jnp.float32)
        xs_rep = _dot(xs, sel_ref[...])
        acc = xs_rep * w
        while acc.shape[1] > fan_out:
            half = acc.shape[1] // 2
            acc = acc[:, :half] + acc[:, half:]
        out_ref[...] = acc
    return body


def _msg(e, xs, eg, eb, es, eq, w0, b0, w1, b1, fan_in, fan_out, tile=2000):
    E, F = e.shape
    fhid = w0.shape[1]
    body = _make_msg_body(E, fan_in, fan_out)
    wide = w1.shape[1]
    return pl.pallas_call(
        body,
        grid=(E // tile,),
        in_specs=[
            pl.BlockSpec((tile, F), lambda i: (i, 0)),
            pl.BlockSpec((tile, fan_in), lambda i: (i, 0)),
            pl.BlockSpec((fan_in, wide), lambda i: (0, 0)),
            pl.BlockSpec((1, F), lambda i: (0, 0)),
            pl.BlockSpec((1, F), lambda i: (0, 0)),
            pl.BlockSpec((1, F), lambda i: (0, 0)),
            pl.BlockSpec((1, F), lambda i: (0, 0)),
            pl.BlockSpec((F, fhid), lambda i: (0, 0)),
            pl.BlockSpec((1, fhid), lambda i: (0, 0)),
            pl.BlockSpec((fhid, wide), lambda i: (0, 0)),
            pl.BlockSpec((1, wide), lambda i: (0, 0)),
        ],
        out_specs=pl.BlockSpec((tile, fan_out), lambda i: (i, 0)),
        out_shape=jax.ShapeDtypeStruct((E, fan_out), jnp.float32),
    )(e, xs, jnp.repeat(jnp.eye(fan_in, dtype=jnp.float32), fan_out, axis=1),
      eg.reshape(1, -1), eb.reshape(1, -1), es, eq,
      w0, b0.reshape(1, -1), w1, b1.reshape(1, -1))


def _make_final_body(E):
    def body(e_ref, hs_ref, hd_ref, eg_ref, eb_ref, s_ref, q_ref,
             w0a_ref, w0b_ref, w0c_ref, b0_ref, w1_ref, b1_ref,
             w2_ref, b2_ref, w3_ref, b3_ref, w4_ref, b4_ref, out_ref):
        mean = s_ref[...] / E
        var = q_ref[...] / E - mean * mean
        en = (e_ref[...] - mean) * (eg_ref[...] * lax.rsqrt(var + 1e-5)) + eb_ref[...]
        t = _leaky(_dot(hs_ref[...], w0a_ref[...])
                   + _dot(hd_ref[...], w0b_ref[...])
                   + _dot(en, w0c_ref[...])
                   + b0_ref[...])
        t = _leaky(_dot(t, w1_ref[...]) + b1_ref[...])
        t = _leaky(_dot(t, w2_ref[...]) + b2_ref[...])
        t = _leaky(_dot(t, w3_ref[...]) + b3_ref[...])
        out_ref[...] = _dot(t, w4_ref[...]) + b4_ref[...]
    return body


def _final_mlp(e, hs, hd, eg, eb, es, eq, w0, b0, w1, b1, w2, b2, w3, b3,
               w4, b4, tile=8000):
    E, F = e.shape
    H = hs.shape[1]
    w0a, w0b, w0c = w0[:H], w0[H:2 * H], w0[2 * H:]
    full = lambda a: pl.BlockSpec(a.shape, lambda i: tuple(0 for _ in a.shape))
    b0r, b1r, b2r, b3r, b4r = (v.reshape(1, -1) for v in (b0, b1, b2, b3, b4))
    args = (e, hs, hd, eg.reshape(1, -1), eb.reshape(1, -1), es, eq,
            w0a, w0b, w0c, b0r, w1, b1r, w2, b2r, w3, b3r, w4, b4r)
    in_specs = [
        pl.BlockSpec((tile, F), lambda i: (i, 0)),
        pl.BlockSpec((tile, H), lambda i: (i, 0)),
        pl.BlockSpec((tile, H), lambda i: (i, 0)),
    ] + [full(a) for a in args[3:]]
    return pl.pallas_call(
        _make_final_body(E),
        grid=(E // tile,),
        in_specs=in_specs,
        out_specs=pl.BlockSpec((tile, 2), lambda i: (i, 0)),
        out_shape=jax.ShapeDtypeStruct((E, 2), jnp.float32),
    )(*args)


def _sc_gather(table, *idxs):
    Nn, D = table.shape
    E = idxs[0].shape[0]
    per_w = E // _NW
    n_full, rem = divmod(per_w, _CH)
    pairs, odd = divmod(n_full, 2)
    n_idx = len(idxs)
    mesh = plsc.VectorSubcoreMesh(core_axis_name="c", subcore_axis_name="s",
                                  num_cores=_NC, num_subcores=_NS)
    scratch = [
        pltpu.VMEM((per_w,), jnp.int32),
        pltpu.VMEM((_CH, D), jnp.float32),
        pltpu.VMEM((_CH, D), jnp.float32),
        pltpu.VMEM((max(rem, 1), D), jnp.float32),
        pltpu.SemaphoreType.DMA,
        pltpu.SemaphoreType.DMA,
        pltpu.SemaphoreType.DMA,
        pltpu.SemaphoreType.DMA,
    ]

    def body(table_ref, *rest):
        idx_refs = rest[:n_idx]
        out_refs = rest[n_idx:2 * n_idx]
        idx_all, buf0, buf1, buf_r, g0, g1, s0, s1 = rest[2 * n_idx:]
        bufs, gsem, ssem = (buf0, buf1), (g0, g1), (s0, s1)
        wid = lax.axis_index("s") * _NC + lax.axis_index("c")
        base = wid * per_w

        for k in range(n_idx):
            out_ref = out_refs[k]

            def drain_store(b, k=k):
                pltpu.make_async_copy(
                    bufs[b], out_refs[k].at[pl.ds(base, _CH)], ssem[b]).wait()

            pltpu.sync_copy(idx_refs[k].at[pl.ds(base, per_w)], idx_all)

            def pair(j2, _, k=k):
                for b in range(2):
                    j = j2 * 2 + b

                    @pl.when(j2 > 0)
                    def _(b=b):
                        drain_store(b)
                    off = base + j * _CH
                    pltpu.async_copy(
                        table_ref.at[idx_all.at[pl.ds(j * _CH, _CH)]],
                        bufs[b], gsem[b]).wait()
                    pltpu.async_copy(bufs[b], out_refs[k].at[pl.ds(off, _CH)],
                                     ssem[b])
                return 0
            lax.fori_loop(0, pairs, pair, 0, unroll=False)
            if odd:
                j = pairs * 2
                if pairs > 0:
                    drain_store(0)
                pltpu.async_copy(
                    table_ref.at[idx_all.at[pl.ds(j * _CH, _CH)]],
                    bufs[0], gsem[0]).wait()
                pltpu.async_copy(bufs[0], out_ref.at[pl.ds(base + j * _CH, _CH)],
                                 ssem[0])
            if rem:
                off = base + n_full * _CH
                pltpu.async_copy(
                    table_ref.at[idx_all.at[pl.ds(n_full * _CH, rem)]],
                    buf_r, gsem[1]).wait()
                pltpu.sync_copy(buf_r, out_ref.at[pl.ds(off, rem)])
            if n_full > 0:
                drain_store(odd)
            if n_full > 1 or (odd and n_full > 0):
                drain_store(1 - odd)

    out_type = tuple(jax.ShapeDtypeStruct((E, D), jnp.float32)
                     for _ in range(n_idx))
    fn = pl.kernel(body, out_type=out_type, mesh=mesh, scratch_types=scratch,
                   compiler_params=pltpu.CompilerParams(use_tc_tiling_on_sc=False))
    return fn(table, *idxs)


def _sc_scatter(msgs, dst, seeds):
    E, D = msgs.shape
    Nn = seeds.shape[1]
    per_w = E // _NW
    n_full, rem = divmod(per_w, _CH)
    rpt = (Nn // _NS) // 8 * 8
    tail = Nn - rpt * _NS
    mesh = plsc.VectorSubcoreMesh(core_axis_name="c", subcore_axis_name="s",
                                  num_cores=_NC, num_subcores=_NS)
    pairs, odd = divmod(n_full, 2)
    scratch = [
        pltpu.VMEM((2, _CH), jnp.int32),
        pltpu.VMEM((_CH, D), jnp.float32),
        pltpu.VMEM((_CH, D), jnp.float32),
        pltpu.VMEM((1, max(rem, 1)), jnp.int32),
        pltpu.VMEM((max(rem, 1), D), jnp.float32),
        pltpu.VMEM_SHARED((Nn, D), jnp.float32),
        pltpu.SemaphoreType.DMA,
        pltpu.SemaphoreType.DMA,
    ]

    def body(m_ref, dst_ref, seeds_ref, out_ref, idx2, buf0, buf1,
             idx_r, buf_r, acc, t0, t1):
        bufs, tsem = (buf0, buf1), (t0, t1)
        cid = lax.axis_index("c")
        sid = lax.axis_index("s")
        wid = sid * _NC + cid
        base = wid * per_w
        pltpu.sync_copy(seeds_ref.at[cid, pl.ds(sid * rpt, rpt)],
                        acc.at[pl.ds(sid * rpt, rpt)])
        if tail:
            @pl.when(sid == 0)
            def _tail_init():
                pltpu.sync_copy(seeds_ref.at[cid, pl.ds(rpt * _NS, tail)],
                                acc.at[pl.ds(rpt * _NS, tail)])
        plsc.subcore_barrier()

        def stage(j, b):
            off = base + j * _CH
            pltpu.async_copy(dst_ref.at[pl.ds(off, _CH)], idx2.at[b], tsem[b])
            pltpu.async_copy(m_ref.at[pl.ds(off, _CH)], bufs[b], tsem[b])

        def wait_stage(b):
            pltpu.make_async_copy(dst_ref.at[pl.ds(base, _CH)], idx2.at[b],
                                  tsem[b]).wait()
            pltpu.make_async_copy(m_ref.at[pl.ds(base, _CH)], bufs[b],
                                  tsem[b]).wait()

        if n_full > 0:
            stage(0, 0)

        def pair(j2, _):
            for b in range(2):
                j = j2 * 2 + b
                nxt = j + 1
                if odd:
                    stage(nxt, 1 - b)
                else:
                    @pl.when(nxt < n_full)
                    def _(nxt=nxt, b=b):
                        stage(nxt, 1 - b)
                wait_stage(b)
                pltpu.sync_copy(bufs[b], acc.at[idx2.at[b]], add=True)
            return 0
        lax.fori_loop(0, pairs, pair, 0, unroll=False)
        if odd:
            wait_stage(0)
            pltpu.sync_copy(bufs[0], acc.at[idx2.at[0]], add=True)
        if rem:
            off = base + n_full * _CH
            pltpu.sync_copy(dst_ref.at[pl.ds(off, rem)], idx_r.at[0])
            pltpu.sync_copy(m_ref.at[pl.ds(off, rem)], buf_r)
            pltpu.sync_copy(buf_r, acc.at[idx_r.at[0]], add=True)
        plsc.subcore_barrier()
        pltpu.sync_copy(acc.at[pl.ds(sid * rpt, rpt)],
                        out_ref.at[cid, pl.ds(sid * rpt, rpt)])
        if tail:
            @pl.when(sid == 0)
            def _tail_out():
                pltpu.sync_copy(acc.at[pl.ds(rpt * _NS, tail)],
                                out_ref.at[cid, pl.ds(rpt * _NS, tail)])

    fn = pl.kernel(body,
                   out_type=jax.ShapeDtypeStruct((_NC, Nn, D), jnp.float32),
                   mesh=mesh, scratch_types=scratch,
                   compiler_params=pltpu.CompilerParams(use_tc_tiling_on_sc=False))
    return fn(msgs, dst, seeds)


def kernel(x, edge_index, e, xbatch,
           bn_node_g, bn_node_b, bn_edge_g, bn_edge_b,
           nn1_W0, nn1_b0, nn1_W1, nn1_b1, conv1_root, conv1_bias,
           nn2_W0, nn2_b0, nn2_W1, nn2_b1, conv2_root, conv2_bias,
           mlp_W0, mlp_b0, mlp_W1, mlp_b1, mlp_W2, mlp_b2,
           mlp_W3, mlp_b3, mlp_W4, mlp_b4):
    src = edge_index[0]
    dst = edge_index[1]

    es, eq = _edge_stats(e)
    xn, r1 = _node1(x, bn_node_g, bn_node_b, conv1_root, conv1_bias)

    (xs,) = _sc_gather(xn, src)
    m1 = _msg(e, xs, bn_edge_g, bn_edge_b, es, eq,
              nn1_W0, nn1_b0, nn1_W1, nn1_b1, fan_in=16, fan_out=32,
              tile=4000)
    seeds1 = jnp.stack([r1, jnp.zeros_like(r1)])
    parts1 = _sc_scatter(m1, dst, seeds1)
    h1, r2 = _node2(parts1, conv2_root, conv2_bias)

    (h1s,) = _sc_gather(h1, src)
    m2 = _msg(e, h1s, bn_edge_g, bn_edge_b, es, eq,
              nn2_W0, nn2_b0, nn2_W1, nn2_b1, fan_in=32, fan_out=64,
              tile=3200)
    seeds2 = jnp.stack([r2, jnp.zeros_like(r2)])
    parts2 = _sc_scatter(m2, dst, seeds2)
    h2 = _hsum(parts2)

    h2s, h2d = _sc_gather(h2, src, dst)
    return _final_mlp(e, h2s, h2d, bn_edge_g, bn_edge_b, es, eq,
                      mlp_W0, mlp_b0, mlp_W1, mlp_b1, mlp_W2, mlp_b2,
                      mlp_W3, mlp_b3, mlp_W4, mlp_b4)


# speedup vs baseline: 2.2664x; 1.0071x over previous
import functools

import jax
import jax.numpy as jnp
from jax import lax
from jax.experimental import pallas as pl
from jax.experimental.pallas import tpu as pltpu
from jax.experimental.pallas import tpu_sc as plsc

_NC, _NS = 2, 16
_NW = _NC * _NS
_CH = 128


def _dot(a, b):
    return jnp.dot(a.astype(jnp.bfloat16), b.astype(jnp.bfloat16),
                   preferred_element_type=jnp.float32)

def _leaky(v):
    return jnp.where(v >= 0, v, 0.1 * v)


def _estats_body(e_ref, s_ref, q_ref):
    i = pl.program_id(0)

    @pl.when(i == 0)
    def _init():
        s_ref[...] = jnp.zeros_like(s_ref)
        q_ref[...] = jnp.zeros_like(q_ref)

    blk = e_ref[...]
    s_ref[...] += jnp.sum(blk, axis=0, keepdims=True)
    q_ref[...] += jnp.sum(blk * blk, axis=0, keepdims=True)


def _edge_stats(e, tile=16000):
    E, F = e.shape
    return pl.pallas_call(
        _estats_body,
        grid=(E // tile,),
        in_specs=[pl.BlockSpec((tile, F), lambda i: (i, 0))],
        out_specs=[pl.BlockSpec((1, F), lambda i: (0, 0))] * 2,
        out_shape=[jax.ShapeDtypeStruct((1, F), jnp.float32)] * 2,
    )(e)


def _node1_body(x_ref, g_ref, b_ref, root_ref, bias_ref, xn_ref, r1_ref):
    x = x_ref[...]
    m = jnp.mean(x, axis=0, keepdims=True)
    var = jnp.mean(x * x, axis=0, keepdims=True) - m * m
    xn = (x - m) * (g_ref[...] * lax.rsqrt(var + 1e-5)) + b_ref[...]
    xn_ref[...] = xn
    r1_ref[...] = (
        _dot(xn, root_ref[...])
        + bias_ref[...]
    )


def _node1(x, g, b, root, bias):
    N, F = x.shape
    Fo = root.shape[1]
    return pl.pallas_call(
        _node1_body,
        out_shape=[
            jax.ShapeDtypeStruct((N, F), jnp.float32),
            jax.ShapeDtypeStruct((N, Fo), jnp.float32),
        ],
    )(x, g.reshape(1, -1), b.reshape(1, -1), root, bias.reshape(1, -1))


def _node2_body(parts_ref, seed_ref, root_ref, bias_ref, h_ref, r_ref):
    h = parts_ref[0] + parts_ref[1] + seed_ref[...]
    h_ref[...] = h
    r_ref[...] = (
        _dot(h, root_ref[...])
        + bias_ref[...]
    )


def _node2(parts, seed, root, bias):
    _, N, F = parts.shape
    Fo = root.shape[1]
    return pl.pallas_call(
        _node2_body,
        out_shape=[
            jax.ShapeDtypeStruct((N, F), jnp.float32),
            jax.ShapeDtypeStruct((N, Fo), jnp.float32),
        ],
    )(parts, seed, root, bias.reshape(1, -1))


def _hsum_body(parts_ref, seed_ref, h_ref):
    h_ref[...] = parts_ref[0] + parts_ref[1] + seed_ref[...]


def _hsum(parts, seed):
    _, N, F = parts.shape
    return pl.pallas_call(
        _hsum_body,
        out_shape=jax.ShapeDtypeStruct((N, F), jnp.float32),
    )(parts, seed)


def _make_msg_body(E, fan_in, fan_out):
    def body(e_ref, xs_ref, sel_ref, eg_ref, eb_ref, s_ref, q_ref,
             w0_ref, b0_ref, w1_ref, b1_ref, out_ref):
        mean = s_ref[...] / E
        var = q_ref[...] / E - mean * mean
        en = (e_ref[...] - mean) * (eg_ref[...] * lax.rsqrt(var + 1e-5)) + eb_ref[...]
        u = _leaky(_dot(en, w0_ref[...]) + b0_ref[...])
        w = _leaky(_dot(u, w1_ref[...]) + b1_ref[...])
        w = w.astype(jnp.bfloat16).astype(jnp.float32)
        xs = xs_ref[...].astype(jnp.bfloat16).astype(jnp.float32)
        xs_rep = _dot(xs, sel_ref[...])
        acc = xs_rep * w
        while acc.shape[1] > fan_out:
            half = acc.shape[1] // 2
            acc = acc[:, :half] + acc[:, half:]
        out_ref[...] = acc
    return body


def _msg(e, xs, eg, eb, es, eq, w0, b0, w1, b1, fan_in, fan_out, tile=2000):
    E, F = e.shape
    fhid = w0.shape[1]
    body = _make_msg_body(E, fan_in, fan_out)
    wide = w1.shape[1]
    return pl.pallas_call(
        body,
        grid=(E // tile,),
        in_specs=[
            pl.BlockSpec((tile, F), lambda i: (i, 0)),
            pl.BlockSpec((tile, fan_in), lambda i: (i, 0)),
            pl.BlockSpec((fan_in, wide), lambda i: (0, 0)),
            pl.BlockSpec((1, F), lambda i: (0, 0)),
            pl.BlockSpec((1, F), lambda i: (0, 0)),
            pl.BlockSpec((1, F), lambda i: (0, 0)),
            pl.BlockSpec((1, F), lambda i: (0, 0)),
            pl.BlockSpec((F, fhid), lambda i: (0, 0)),
            pl.BlockSpec((1, fhid), lambda i: (0, 0)),
            pl.BlockSpec((fhid, wide), lambda i: (0, 0)),
            pl.BlockSpec((1, wide), lambda i: (0, 0)),
        ],
        out_specs=pl.BlockSpec((tile, fan_out), lambda i: (i, 0)),
        out_shape=jax.ShapeDtypeStruct((E, fan_out), jnp.float32),
    )(e, xs, jnp.repeat(jnp.eye(fan_in, dtype=jnp.float32), fan_out, axis=1),
      eg.reshape(1, -1), eb.reshape(1, -1), es, eq,
      w0, b0.reshape(1, -1), w1, b1.reshape(1, -1))


def _make_final_body(E):
    def body(e_ref, hs_ref, hd_ref, eg_ref, eb_ref, s_ref, q_ref,
             w0a_ref, w0b_ref, w0c_ref, b0_ref, w1_ref, b1_ref,
             w2_ref, b2_ref, w3_ref, b3_ref, w4_ref, b4_ref, out_ref):
        mean = s_ref[...] / E
        var = q_ref[...] / E - mean * mean
        en = (e_ref[...] - mean) * (eg_ref[...] * lax.rsqrt(var + 1e-5)) + eb_ref[...]
        t = _leaky(_dot(hs_ref[...], w0a_ref[...])
                   + _dot(hd_ref[...], w0b_ref[...])
                   + _dot(en, w0c_ref[...])
                   + b0_ref[...])
        t = _leaky(_dot(t, w1_ref[...]) + b1_ref[...])
        t = _leaky(_dot(t, w2_ref[...]) + b2_ref[...])
        t = _leaky(_dot(t, w3_ref[...]) + b3_ref[...])
        out_ref[...] = _dot(t, w4_ref[...]) + b4_ref[...]
    return body


def _final_mlp(e, hs, hd, eg, eb, es, eq, w0, b0, w1, b1, w2, b2, w3, b3,
               w4, b4, tile=8000):
    E, F = e.shape
    H = hs.shape[1]
    w0a, w0b, w0c = w0[:H], w0[H:2 * H], w0[2 * H:]
    full = lambda a: pl.BlockSpec(a.shape, lambda i: tuple(0 for _ in a.shape))
    b0r, b1r, b2r, b3r, b4r = (v.reshape(1, -1) for v in (b0, b1, b2, b3, b4))
    args = (e, hs, hd, eg.reshape(1, -1), eb.reshape(1, -1), es, eq,
            w0a, w0b, w0c, b0r, w1, b1r, w2, b2r, w3, b3r, w4, b4r)
    in_specs = [
        pl.BlockSpec((tile, F), lambda i: (i, 0)),
        pl.BlockSpec((tile, H), lambda i: (i, 0)),
        pl.BlockSpec((tile, H), lambda i: (i, 0)),
    ] + [full(a) for a in args[3:]]
    return pl.pallas_call(
        _make_final_body(E),
        grid=(E // tile,),
        in_specs=in_specs,
        out_specs=pl.BlockSpec((tile, 2), lambda i: (i, 0)),
        out_shape=jax.ShapeDtypeStruct((E, 2), jnp.float32),
    )(*args)


def _sc_gather(table, *idxs):
    Nn, D = table.shape
    E = idxs[0].shape[0]
    per_w = E // _NW
    n_full, rem = divmod(per_w, _CH)
    pairs, odd = divmod(n_full, 2)
    n_idx = len(idxs)
    mesh = plsc.VectorSubcoreMesh(core_axis_name="c", subcore_axis_name="s",
                                  num_cores=_NC, num_subcores=_NS)
    scratch = [
        pltpu.VMEM((per_w,), jnp.int32),
        pltpu.VMEM((_CH, D), jnp.float32),
        pltpu.VMEM((_CH, D), jnp.float32),
        pltpu.VMEM((max(rem, 1), D), jnp.float32),
        pltpu.SemaphoreType.DMA,
        pltpu.SemaphoreType.DMA,
        pltpu.SemaphoreType.DMA,
        pltpu.SemaphoreType.DMA,
    ]

    def body(table_ref, *rest):
        idx_refs = rest[:n_idx]
        out_refs = rest[n_idx:2 * n_idx]
        idx_all, buf0, buf1, buf_r, g0, g1, s0, s1 = rest[2 * n_idx:]
        bufs, gsem, ssem = (buf0, buf1), (g0, g1), (s0, s1)
        wid = lax.axis_index("s") * _NC + lax.axis_index("c")
        base = wid * per_w

        for k in range(n_idx):
            out_ref = out_refs[k]

            def drain_store(b, k=k):
                pltpu.make_async_copy(
                    bufs[b], out_refs[k].at[pl.ds(base, _CH)], ssem[b]).wait()

            pltpu.sync_copy(idx_refs[k].at[pl.ds(base, per_w)], idx_all)

            def pair(j2, _, k=k):
                for b in range(2):
                    j = j2 * 2 + b

                    @pl.when(j2 > 0)
                    def _(b=b):
                        drain_store(b)
                    off = base + j * _CH
                    pltpu.async_copy(
                        table_ref.at[idx_all.at[pl.ds(j * _CH, _CH)]],
                        bufs[b], gsem[b]).wait()
                    pltpu.async_copy(bufs[b], out_refs[k].at[pl.ds(off, _CH)],
                                     ssem[b])
                return 0
            lax.fori_loop(0, pairs, pair, 0, unroll=False)
            if odd:
                j = pairs * 2
                if pairs > 0:
                    drain_store(0)
                pltpu.async_copy(
                    table_ref.at[idx_all.at[pl.ds(j * _CH, _CH)]],
                    bufs[0], gsem[0]).wait()
                pltpu.async_copy(bufs[0], out_ref.at[pl.ds(base + j * _CH, _CH)],
                                 ssem[0])
            if rem:
                off = base + n_full * _CH
                pltpu.async_copy(
                    table_ref.at[idx_all.at[pl.ds(n_full * _CH, rem)]],
                    buf_r, gsem[1]).wait()
                pltpu.sync_copy(buf_r, out_ref.at[pl.ds(off, rem)])
            if n_full > 0:
                drain_store(odd)
            if n_full > 1 or (odd and n_full > 0):
                drain_store(1 - odd)

    out_type = tuple(jax.ShapeDtypeStruct((E, D), jnp.float32)
                     for _ in range(n_idx))
    fn = pl.kernel(body, out_type=out_type, mesh=mesh, scratch_types=scratch,
                   compiler_params=pltpu.CompilerParams(use_tc_tiling_on_sc=False))
    return fn(table, *idxs)


def _sc_scatter(msgs, dst, Nn):
    E, D = msgs.shape
    per_w = E // _NW
    n_full, rem = divmod(per_w, _CH)
    rpt = (Nn // _NS) // 8 * 8
    tail = Nn - rpt * _NS
    mesh = plsc.VectorSubcoreMesh(core_axis_name="c", subcore_axis_name="s",
                                  num_cores=_NC, num_subcores=_NS)
    pairs, odd = divmod(n_full, 2)
    scratch = [
        pltpu.VMEM((2, _CH), jnp.int32),
        pltpu.VMEM((_CH, D), jnp.float32),
        pltpu.VMEM((_CH, D), jnp.float32),
        pltpu.VMEM((1, max(rem, 1)), jnp.int32),
        pltpu.VMEM((max(rem, 1), D), jnp.float32),
        pltpu.VMEM_SHARED((Nn, D), jnp.float32),
        pltpu.VMEM((_CH, D), jnp.float32),
        pltpu.SemaphoreType.DMA,
        pltpu.SemaphoreType.DMA,
    ]

    def body(m_ref, dst_ref, zb_ref, out_ref, idx2, buf0, buf1,
             idx_r, buf_r, acc, zbuf, t0, t1):
        bufs, tsem = (buf0, buf1), (t0, t1)
        cid = lax.axis_index("c")
        sid = lax.axis_index("s")
        wid = sid * _NC + cid
        base = wid * per_w
        pltpu.sync_copy(zb_ref, zbuf)
        zoff = 0
        while zoff < rpt:
            zn = min(_CH, rpt - zoff)
            pltpu.sync_copy(zbuf.at[pl.ds(0, zn)],
                            acc.at[pl.ds(sid * rpt + zoff, zn)])
            zoff += zn
        if tail:
            @pl.when(sid == 0)
            def _tail_init():
                pltpu.sync_copy(zbuf.at[pl.ds(0, tail)],
                                acc.at[pl.ds(rpt * _NS, tail)])
        plsc.subcore_barrier()

        def stage(j, b):
            off = base + j * _CH
            pltpu.async_copy(dst_ref.at[pl.ds(off, _CH)], idx2.at[b], tsem[b])
            pltpu.async_copy(m_ref.at[pl.ds(off, _CH)], bufs[b], tsem[b])

        def wait_stage(b):
            pltpu.make_async_copy(dst_ref.at[pl.ds(base, _CH)], idx2.at[b],
                                  tsem[b]).wait()
            pltpu.make_async_copy(m_ref.at[pl.ds(base, _CH)], bufs[b],
                                  tsem[b]).wait()

        if n_full > 0:
            stage(0, 0)

        def pair(j2, _):
            for b in range(2):
                j = j2 * 2 + b
                nxt = j + 1
                if odd:
                    stage(nxt, 1 - b)
                else:
                    @pl.when(nxt < n_full)
                    def _(nxt=nxt, b=b):
                        stage(nxt, 1 - b)
                wait_stage(b)
                pltpu.sync_copy(bufs[b], acc.at[idx2.at[b]], add=True)
            return 0
        lax.fori_loop(0, pairs, pair, 0, unroll=False)
        if odd:
            wait_stage(0)
            pltpu.sync_copy(bufs[0], acc.at[idx2.at[0]], add=True)
        if rem:
            off = base + n_full * _CH
            pltpu.sync_copy(dst_ref.at[pl.ds(off, rem)], idx_r.at[0])
            pltpu.sync_copy(m_ref.at[pl.ds(off, rem)], buf_r)
            pltpu.sync_copy(buf_r, acc.at[idx_r.at[0]], add=True)
        plsc.subcore_barrier()
        pltpu.sync_copy(acc.at[pl.ds(sid * rpt, rpt)],
                        out_ref.at[cid, pl.ds(sid * rpt, rpt)])
        if tail:
            @pl.when(sid == 0)
            def _tail_out():
                pltpu.sync_copy(acc.at[pl.ds(rpt * _NS, tail)],
                                out_ref.at[cid, pl.ds(rpt * _NS, tail)])

    fn = pl.kernel(body,
                   out_type=jax.ShapeDtypeStruct((_NC, Nn, D), jnp.float32),
                   mesh=mesh, scratch_types=scratch,
                   compiler_params=pltpu.CompilerParams(use_tc_tiling_on_sc=False))
    return fn(msgs, dst, jnp.zeros((_CH, D), jnp.float32))


def kernel(x, edge_index, e, xbatch,
           bn_node_g, bn_node_b, bn_edge_g, bn_edge_b,
           nn1_W0, nn1_b0, nn1_W1, nn1_b1, conv1_root, conv1_bias,
           nn2_W0, nn2_b0, nn2_W1, nn2_b1, conv2_root, conv2_bias,
           mlp_W0, mlp_b0, mlp_W1, mlp_b1, mlp_W2, mlp_b2,
           mlp_W3, mlp_b3, mlp_W4, mlp_b4):
    src = edge_index[0]
    dst = edge_index[1]

    es, eq = _edge_stats(e)
    xn, r1 = _node1(x, bn_node_g, bn_node_b, conv1_root, conv1_bias)

    (xs,) = _sc_gather(xn, src)
    m1 = _msg(e, xs, bn_edge_g, bn_edge_b, es, eq,
              nn1_W0, nn1_b0, nn1_W1, nn1_b1, fan_in=16, fan_out=32,
              tile=4000)
    parts1 = _sc_scatter(m1, dst, x.shape[0])
    h1, r2 = _node2(parts1, r1, conv2_root, conv2_bias)

    (h1s,) = _sc_gather(h1, src)
    m2 = _msg(e, h1s, bn_edge_g, bn_edge_b, es, eq,
              nn2_W0, nn2_b0, nn2_W1, nn2_b1, fan_in=32, fan_out=64,
              tile=3200)
    parts2 = _sc_scatter(m2, dst, x.shape[0])
    h2 = _hsum(parts2, r2)

    h2s, h2d = _sc_gather(h2, src, dst)
    return _final_mlp(e, h2s, h2d, bn_edge_g, bn_edge_b, es, eq,
                      mlp_W0, mlp_b0, mlp_W1, mlp_b1, mlp_W2, mlp_b2,
                      mlp_W3, mlp_b3, mlp_W4, mlp_b4)
